# baseline (speedup 1.0000x reference)
import functools

import jax
import jax.numpy as jnp
from jax import lax
from jax.experimental import pallas as pl
from jax.experimental.pallas import tpu as pltpu
from jax.experimental.pallas import tpu_sc as plsc

ALPHA = 0.2
EPS = 1e-16
NEG = -1e9

NC = 2
NS = 16
NW = NC * NS
L = 16

_MESH = dict(core_axis_name="c", subcore_axis_name="s", num_cores=NC,
             num_subcores=NS)


def _cdiv(a, b):
    return (a + b - 1) // b


def _mm_hsd_body(num_ref, den_ref, w_ref, asd_ref, h_ref, hsd_ref, *, layer2):
    x = num_ref[...]
    if layer2:
        x = x / (den_ref[...] + EPS)
        x = jnp.where(x > 0, x, jnp.exp(x) - 1.0)
    h = jnp.dot(x, w_ref[...], preferred_element_type=jnp.float32)
    h_ref[...] = h
    hsd_ref[...] = jnp.dot(h, asd_ref[...], preferred_element_type=jnp.float32)


def _mm_hsd(x, den, W, a_s, a_d, *, layer2, blk=400):
    n, k = x.shape
    asd = jnp.stack([a_s, a_d], axis=1)
    kfn = functools.partial(_mm_hsd_body, layer2=layer2)
    return pl.pallas_call(
        kfn,
        grid=(n // blk,),
        in_specs=[
            pl.BlockSpec((blk, k), lambda i: (i, 0)),
            pl.BlockSpec((blk, 1), lambda i: (i, 0)),
            pl.BlockSpec((k, 128), lambda i: (0, 0)),
            pl.BlockSpec((128, 2), lambda i: (0, 0)),
        ],
        out_specs=[
            pl.BlockSpec((blk, 128), lambda i: (i, 0)),
            pl.BlockSpec((blk, 2), lambda i: (i, 0)),
        ],
        out_shape=[
            jax.ShapeDtypeStruct((n, 128), jnp.float32),
            jax.ShapeDtypeStruct((n, 2), jnp.float32),
        ],
    )(x, den, W, asd)


def _final_graph_body(num_ref, den_ref, w_ref, proj_ref, x_out_ref, psum_ref):
    i = pl.program_id(0)
    x = num_ref[...] / (den_ref[...] + EPS)
    x = jnp.where(x > 0, x, jnp.exp(x) - 1.0)
    x_out_ref[...] = x
    u = jnp.tanh(jnp.dot(x, w_ref[...], preferred_element_type=jnp.float32))
    p = jnp.dot(u, proj_ref[...], preferred_element_type=jnp.float32)

    @pl.when(i == 0)
    def _init():
        psum_ref[...] = jnp.zeros_like(psum_ref)

    psum_ref[...] = psum_ref[...] + jnp.sum(p)


def _final_graph(num, den, weight_W, weight_proj, *, blk=400):
    n = num.shape[0]
    return pl.pallas_call(
        _final_graph_body,
        grid=(n // blk,),
        in_specs=[
            pl.BlockSpec((blk, 128), lambda i: (i, 0)),
            pl.BlockSpec((blk, 1), lambda i: (i, 0)),
            pl.BlockSpec((128, 128), lambda i: (0, 0)),
            pl.BlockSpec((128, 1), lambda i: (0, 0)),
        ],
        out_specs=[
            pl.BlockSpec((blk, 128), lambda i: (i, 0)),
            pl.BlockSpec((1, 1), lambda i: (0, 0)),
        ],
        out_shape=[
            jax.ShapeDtypeStruct((n, 128), jnp.float32),
            jax.ShapeDtypeStruct((1, 1), jnp.float32),
        ],
    )(num, den, weight_W, weight_proj)


def _logits_body(g1_ref, g2_ref, a_ref, w_ref, b_ref, out_ref):
    a0 = a_ref[0]
    a1 = a_ref[1]
    of = a0 * g1_ref[...] + a1 * g2_ref[...]
    z = jnp.dot(of, w_ref[...], preferred_element_type=jnp.float32) + b_ref[...]
    m = jnp.max(z, axis=1, keepdims=True)
    z = z - m
    out_ref[...] = z - jnp.log(jnp.sum(jnp.exp(z), axis=1, keepdims=True))


def _logits(g1, g2, att, w_pad, b_pad, *, blk=512):
    b = g1.shape[0]
    return pl.pallas_call(
        _logits_body,
        grid=(b // blk,),
        in_specs=[
            pl.BlockSpec((blk, 128), lambda i: (i, 0)),
            pl.BlockSpec((blk, 128), lambda i: (i, 0)),
            pl.BlockSpec(memory_space=pltpu.SMEM),
            pl.BlockSpec((128, 128), lambda i: (0, 0)),
            pl.BlockSpec((1, 128), lambda i: (0, 0)),
        ],
        out_specs=pl.BlockSpec((blk, 128), lambda i: (i, 0)),
        out_shape=jax.ShapeDtypeStruct((b, 128), jnp.float32),
    )(g1, g2, att, w_pad, b_pad)


def _embed_mean(word_emb_pad, feat_flat, n, words, kcols, kpad):
    tb = n // L
    nb = _cdiv(tb, NW)
    cv = kcols // L

    @functools.partial(
        pl.kernel,
        out_type=jax.ShapeDtypeStruct((n, kpad), jnp.float32),
        mesh=plsc.VectorSubcoreMesh(**_MESH),
        compiler_params=pltpu.CompilerParams(needs_layout_passes=False),
        scratch_types=[
            pltpu.VMEM((L * words,), jnp.int32),
            pltpu.VMEM((L * words,), jnp.int32),
            pltpu.VMEM((2, L * words, kcols), jnp.float32),
            pltpu.VMEM((L, kpad), jnp.float32),
            pltpu.SemaphoreType.DMA,
            pltpu.SemaphoreType.DMA,
        ],
    )
    def k(we_hbm, fi_hbm, out_hbm, fidx_a, fidx_b, rows_v, obuf_v, sem_a, sem_b):
        wid = lax.axis_index("s") * NC + lax.axis_index("c")

        def jof(b):
            return jnp.minimum(wid * nb + b, tb - 1)

        def load_issue(b, buf, sem):
            j = jof(b)
            fidx = fidx_a if buf == 0 else fidx_b
            pltpu.sync_copy(fi_hbm.at[pl.ds(j * (L * words), L * words)],
                            fidx)
            return pltpu.async_copy(we_hbm.at[fidx], rows_v.at[buf], sem)

        def compute_out(b, buf):
            def node(kk, _):
                base = kk * words
                for c in range(cv):
                    a = rows_v[buf, base, pl.ds(c * L, L)]
                    for r in range(1, words):
                        a = a + rows_v[buf, base + r, pl.ds(c * L, L)]
                    obuf_v[kk, pl.ds(c * L, L)] = a * (1.0 / words)
                return 0

            lax.fori_loop(0, L, node, 0)
            pltpu.sync_copy(obuf_v, out_hbm.at[pl.ds(jof(b) * L, L)])

        load_issue(0, 0, sem_a)

        def pair(i, _):
            ba = 2 * i
            bb = ba + 1

            @pl.when(bb < nb)
            def _ib():
                load_issue(bb, 1, sem_b)

            pltpu.make_async_copy(we_hbm.at[fidx_a], rows_v.at[0],
                                  sem_a).wait()
            compute_out(ba, 0)

            @pl.when(ba + 2 < nb)
            def _ia():
                load_issue(ba + 2, 0, sem_a)

            @pl.when(bb < nb)
            def _cb():
                pltpu.make_async_copy(we_hbm.at[fidx_b], rows_v.at[1],
                                      sem_b).wait()
                compute_out(bb, 1)

            return 0

        lax.fori_loop(0, _cdiv(nb, 2), pair, 0)

    return k(word_emb_pad, feat_flat)


def _edge_aggregate(h, hsd, src, dst, n):
    e = src.shape[0]
    chunk = 2048
    n_pad = _cdiv(n, 32) * 32
    e_pad = _cdiv(e, NS * chunk) * (NS * chunk)
    eb = 48
    epc = e_pad // NS
    nq = n_pad // 4
    nqp = nq
    ndr = _cdiv(_cdiv(nq + 1, 128), L) * L
    q = _cdiv(_cdiv(nq, NS), 8) * 8
    zc = _cdiv(nqp, NS * eb)
    z = zc * eb
    dq = 8

    src_p = jnp.pad(src, (0, e_pad - e))
    dst_p = jnp.pad(dst, (0, e_pad - e), constant_values=n_pad)
    if n_pad == n:
        h_p = h
        hs = hsd[:, 0:1].reshape(n)
        hd = hsd[:, 1:2].reshape(n)
    else:
        h_p = jnp.pad(h, ((0, n_pad - n), (0, 0)))
        hs = jnp.pad(hsd[:, 0:1].reshape(n), (0, n_pad - n))
        hd = jnp.pad(hsd[:, 1:2].reshape(n), (0, n_pad - n))

    @functools.partial(
        pl.kernel,
        out_type=[
            jax.ShapeDtypeStruct((n_pad, 128), jnp.float32),
            jax.ShapeDtypeStruct((2, NC, ndr, 128), jnp.float32),
        ],
        mesh=plsc.VectorSubcoreMesh(**_MESH),
        compiler_params=pltpu.CompilerParams(needs_layout_passes=False),
        scratch_types=[
            pltpu.VMEM((n_pad,), jnp.float32),
            pltpu.VMEM((nq,), jnp.float32),
            pltpu.VMEM((ndr, 128), jnp.float32),
            pltpu.VMEM((ndr,), jnp.int32),
            pltpu.VMEM((chunk,), jnp.int32),
            pltpu.VMEM((chunk,), jnp.int32),
            pltpu.VMEM((chunk + eb,), jnp.int32),
            pltpu.VMEM((chunk + eb,), jnp.int32),
            pltpu.VMEM((2, eb), jnp.int32),
            pltpu.VMEM((2, eb), jnp.int32),
            pltpu.VMEM((2, eb), jnp.float32),
            pltpu.VMEM((2, eb, 128), jnp.float32),
            pltpu.VMEM_SHARED((nqp, 128), jnp.float32),
            pltpu.VMEM_SHARED((ndr, 128), jnp.float32),
            pltpu.SemaphoreType.DMA,
            pltpu.SemaphoreType.DMA,
            pltpu.SemaphoreType.DMA,
        ],
    )
    def k(h_hbm, hs_hbm, hd_hbm, src_hbm, dst_hbm, num_hbm, den_hbm,
          hs_v, hdq_v, den_v, iden_v, src_v, dst_v, sc_v, dc_v,
          gidx2_v, dloc2_v, w_v, rows_v, acc_spm, denacc_spm, sem_ga,
          sem_gb, sem_s):
        cid = lax.axis_index("c")
        sid = lax.axis_index("s")
        zero = jnp.zeros((L,), jnp.float32)
        ramp = lax.iota(jnp.int32, L)

        pltpu.sync_copy(hs_hbm, hs_v)

        for i in range(ndr // L):
            iden_v[pl.ds(i * L, L)] = ramp + (i * L)

        def phase_body(ph, _):
            base = pl.multiple_of(ph * (2 * nq) + cid * nq, 8)
            pltpu.sync_copy(hd_hbm.at[pl.ds(base, nq)], hdq_v)

            def zden(i, _):
                for c in range(8):
                    den_v[i, pl.ds(c * L, L)] = zero
                return 0

            lax.fori_loop(0, ndr, zden, 0)

            def zrow(r, _):
                for c in range(8):
                    rows_v[0, r, pl.ds(c * L, L)] = zero
                return 0

            lax.fori_loop(0, eb, zrow, 0)

            zstart = jnp.minimum(sid * z, nqp - z)

            def zacc(i, _):
                pltpu.sync_copy(rows_v.at[0], acc_spm.at[pl.ds(zstart + i * eb, eb)])
                return 0

            lax.fori_loop(0, zc, zacc, 0)

            @pl.when(sid == 0)
            def _zden_shared():
                pltpu.sync_copy(den_v, denacc_spm)

            plsc.subcore_barrier()

            def chunk_body(ci, _):
                eoff = sid * epc + ci * chunk
                pltpu.sync_copy(src_hbm.at[pl.ds(eoff, chunk)], src_v)
                pltpu.sync_copy(dst_hbm.at[pl.ds(eoff, chunk)], dst_v)

                def pre(i, _):
                    sc_v[pl.ds(i * L, L)] = ramp * 0
                    dc_v[pl.ds(i * L, L)] = ramp * 0 + nq
                    return 0

                lax.fori_loop(0, (chunk + eb) // L, pre, 0)

                def cmp_body(j, cnt):
                    s = src_v[pl.ds(j * L, L)]
                    d = dst_v[pl.ds(j * L, L)]
                    inq = (d >= base) & (d < base + nq)
                    dloc = d - base
                    csum = plsc.cumsum(inq.astype(jnp.int32))
                    idx = cnt + csum - 1
                    plsc.store_scatter(sc_v, [idx], s, mask=inq)
                    plsc.store_scatter(dc_v, [idx], dloc, mask=inq)
                    return cnt + csum[L - 1]

                cnt = lax.fori_loop(0, chunk // L, cmp_body, 0)
                nblk = lax.div(cnt + (eb - 1), eb)

                def prep(b, buf):
                    boff = b * eb
                    for j in range(eb // L):
                        s = sc_v[pl.ds(boff + j * L, L)]
                        dl = dc_v[pl.ds(boff + j * L, L)]
                        valid = dl < nq
                        dl = jnp.where(valid, dl, 0)
                        gidx2_v[buf, pl.ds(j * L, L)] = s
                        dloc2_v[buf, pl.ds(j * L, L)] = dl
                        hs16 = plsc.load_gather(hs_v, [s])
                        hd16 = plsc.load_gather(hdq_v, [dl])
                        ee = hs16 + hd16
                        w = jnp.exp(jnp.where(ee >= 0, ee, ALPHA * ee))
                        w = jnp.where(valid, w, 0.0)
                        rr = lax.shift_right_logical(dl, 7)
                        cc = lax.bitwise_and(dl, 127)
                        plsc.addupdate_scatter(den_v, [rr, cc], w)
                        w_v[buf, pl.ds(j * L, L)] = w

                def issue(buf, sem):
                    return pltpu.async_copy(h_hbm.at[gidx2_v.at[buf]],
                                            rows_v.at[buf], sem)

                def scale_scatter(buf):
                    for g in range(eb // L):
                        wv = w_v[buf, pl.ds(g * L, L)]
                        for lane in range(L):
                            wr = wv[lane]
                            r = g * L + lane
                            for c in range(8):
                                rows_v[buf, r, pl.ds(c * L, L)] = (
                                    rows_v[buf, r, pl.ds(c * L, L)] * wr)
                    pltpu.async_copy(rows_v.at[buf],
                                     acc_spm.at[dloc2_v.at[buf]],
                                     sem_s, add=True).wait()

                @pl.when(nblk > 0)
                def _prologue():
                    prep(0, 0)
                    issue(0, sem_ga)

                def pair_body(i, _):
                    ba = 2 * i
                    bb = ba + 1

                    @pl.when(bb < nblk)
                    def _pb():
                        prep(bb, 1)
                        issue(1, sem_gb)

                    pltpu.make_async_copy(h_hbm.at[gidx2_v.at[0]],
                                          rows_v.at[0], sem_ga).wait()
                    scale_scatter(0)

                    @pl.when(ba + 2 < nblk)
                    def _pa():
                        prep(ba + 2, 0)
                        issue(0, sem_ga)

                    @pl.when(bb < nblk)
                    def _cb():
                        pltpu.make_async_copy(h_hbm.at[gidx2_v.at[1]],
                                              rows_v.at[1], sem_gb).wait()
                        scale_scatter(1)

                    return 0

                lax.fori_loop(0, lax.shift_right_logical(nblk + 1, 1),
                              pair_body, 0)
                return 0

            lax.fori_loop(0, epc // chunk, chunk_body, 0)

            pltpu.async_copy(den_v, denacc_spm.at[iden_v], sem_s,
                             add=True).wait()
            plsc.subcore_barrier()

            rstart = jnp.minimum(sid * q, nq - q)
            pltpu.sync_copy(acc_spm.at[pl.ds(rstart, q)],
                            num_hbm.at[pl.ds(base + rstart, q)])
            dstart = jnp.minimum(sid * dq, ndr - dq)
            pltpu.sync_copy(denacc_spm.at[pl.ds(dstart, dq)],
                            den_hbm.at[ph, cid, pl.ds(dstart, dq)])
            plsc.subcore_barrier()
            return 0

        lax.fori_loop(0, 2, phase_body, 0)

    num_p, den_p = k(h_p, hs, hd, src_p, dst_p)
    num = num_p[:n]
    den = den_p.reshape(2 * NC, ndr * 128)[:, :nq].reshape(-1)[:n]
    return num, den.reshape(n, 1)


def _pair_gather(x1, x2, idx1, idx2):
    b = idx1.shape[0]
    r = b // NW

    @functools.partial(
        pl.kernel,
        out_type=[
            jax.ShapeDtypeStruct((b, 128), jnp.float32),
            jax.ShapeDtypeStruct((b, 128), jnp.float32),
        ],
        mesh=plsc.VectorSubcoreMesh(**_MESH),
        compiler_params=pltpu.CompilerParams(needs_layout_passes=False),
        scratch_types=[
            pltpu.VMEM((r,), jnp.int32),
            pltpu.VMEM((r, 128), jnp.float32),
            pltpu.SemaphoreType.DMA,
        ],
    )
    def k(x1_hbm, x2_hbm, i1_hbm, i2_hbm, o1_hbm, o2_hbm, idx_v, rows_v, sem):
        wid = lax.axis_index("s") * NC + lax.axis_index("c")
        base = wid * r
        pltpu.sync_copy(i1_hbm.at[pl.ds(base, r)], idx_v)
        pltpu.async_copy(x1_hbm.at[idx_v], rows_v, sem).wait()
        pltpu.sync_copy(rows_v, o1_hbm.at[pl.ds(base, r)])
        pltpu.sync_copy(i2_hbm.at[pl.ds(base, r)], idx_v)
        pltpu.async_copy(x2_hbm.at[idx_v], rows_v, sem).wait()
        pltpu.sync_copy(rows_v, o2_hbm.at[pl.ds(base, r)])

    return k(x1, x2, idx1, idx2)


def _spgat(x, den0, src, dst, W1p, a1s, a1d, W2, a2s, a2d, n, weight_W,
           weight_proj):
    h1, hsd1 = _mm_hsd(x, den0, W1p, a1s, a1d, layer2=False)
    num1, den1 = _edge_aggregate(h1, hsd1, src, dst, n)
    h2, hsd2 = _mm_hsd(num1, den1, W2, a2s, a2d, layer2=True)
    num2, den2 = _edge_aggregate(h2, hsd2, src, dst, n)
    return _final_graph(num2, den2, weight_W, weight_proj)


def kernel(feat_idx, tw_src, tw_dst, ut_src, ut_dst, tw_graph_idx, ut_graph_idx,
           word_embedding, user_embedding, tw_W1, tw_a1s, tw_a1d, tw_W2, tw_a2s,
           tw_a2d, ut_W1, ut_a1s, ut_a1d, ut_W2, ut_a2s, ut_a2d, weight_W,
           weight_proj, out_W, out_b):
    n_tw, words = feat_idx.shape
    n_ut = user_embedding.shape[0]
    nfeat = word_embedding.shape[1]
    kpad = 384
    kcols = kpad

    we_pad = jnp.pad(word_embedding, ((0, 0), (0, kcols - nfeat)))
    feat_flat = feat_idx.reshape(-1).astype(jnp.int32)
    twt_Xp = _embed_mean(we_pad, feat_flat, n_tw, words, kcols, kpad)
    ue_p = jnp.pad(user_embedding, ((0, 0), (0, kpad - nfeat)))
    tw_W1p = jnp.pad(tw_W1, ((0, kpad - nfeat), (0, 0)))
    ut_W1p = jnp.pad(ut_W1, ((0, kpad - nfeat), (0, 0)))

    one_tw = jnp.ones((n_tw, 1), jnp.float32)
    one_ut = jnp.ones((n_ut, 1), jnp.float32)
    tw_X, tw_psum = _spgat(twt_Xp, one_tw, tw_src.astype(jnp.int32),
                           tw_dst.astype(jnp.int32), tw_W1p, tw_a1s, tw_a1d,
                           tw_W2, tw_a2s, tw_a2d, n_tw, weight_W, weight_proj)
    tu_X, tu_psum = _spgat(ue_p, one_ut, ut_src.astype(jnp.int32),
                           ut_dst.astype(jnp.int32), ut_W1p, ut_a1s, ut_a1d,
                           ut_W2, ut_a2s, ut_a2d, n_ut, weight_W, weight_proj)

    att_tw = tw_psum[0, 0] / n_tw
    att_tu = tu_psum[0, 0] / n_ut
    m = jnp.maximum(att_tw, att_tu)
    e0 = jnp.exp(att_tw - m)
    e1 = jnp.exp(att_tu - m)
    att = jnp.stack([e0, e1]) / (e0 + e1)

    g1, g2 = _pair_gather(tw_X, tu_X, tw_graph_idx.astype(jnp.int32),
                          ut_graph_idx.astype(jnp.int32))

    nclass = out_W.shape[0]
    w_pad = jnp.pad(out_W.T, ((0, 0), (0, 128 - nclass)))
    b_pad = jnp.pad(out_b[None, :], ((0, 0), (0, 128 - nclass)),
                    constant_values=NEG)
    lp = _logits(g1, g2, att, w_pad, b_pad)
    return lp[:, :nclass]


# speedup vs baseline: 1.0015x; 1.0015x over previous
import functools

import jax
import jax.numpy as jnp
from jax import lax
from jax.experimental import pallas as pl
from jax.experimental.pallas import tpu as pltpu
from jax.experimental.pallas import tpu_sc as plsc

ALPHA = 0.2
EPS = 1e-16
NEG = -1e9

NC = 2
NS = 16
NW = NC * NS
L = 16

_MESH = dict(core_axis_name="c", subcore_axis_name="s", num_cores=NC,
             num_subcores=NS)


def _cdiv(a, b):
    return (a + b - 1) // b


def _mm_hsd_body(num_ref, den_ref, w_ref, asd_ref, h_ref, hsd_ref, *, layer2):
    x = num_ref[...]
    if layer2:
        x = x / (den_ref[...] + EPS)
        x = jnp.where(x > 0, x, jnp.exp(x) - 1.0)
    h = jnp.dot(x, w_ref[...], preferred_element_type=jnp.float32)
    h_ref[...] = h
    hsd_ref[...] = jnp.dot(h, asd_ref[...], preferred_element_type=jnp.float32)


def _mm_hsd(x, den, W, a_s, a_d, *, layer2, blk=400):
    n, k = x.shape
    asd = jnp.stack([a_s, a_d], axis=1)
    kfn = functools.partial(_mm_hsd_body, layer2=layer2)
    return pl.pallas_call(
        kfn,
        grid=(n // blk,),
        in_specs=[
            pl.BlockSpec((blk, k), lambda i: (i, 0)),
            pl.BlockSpec((blk, 1), lambda i: (i, 0)),
            pl.BlockSpec((k, 128), lambda i: (0, 0)),
            pl.BlockSpec((128, 2), lambda i: (0, 0)),
        ],
        out_specs=[
            pl.BlockSpec((blk, 128), lambda i: (i, 0)),
            pl.BlockSpec((blk, 2), lambda i: (i, 0)),
        ],
        out_shape=[
            jax.ShapeDtypeStruct((n, 128), jnp.float32),
            jax.ShapeDtypeStruct((n, 2), jnp.float32),
        ],
    )(x, den, W, asd)


def _final_graph_body(num_ref, den_ref, w_ref, proj_ref, x_out_ref, psum_ref):
    i = pl.program_id(0)
    x = num_ref[...] / (den_ref[...] + EPS)
    x = jnp.where(x > 0, x, jnp.exp(x) - 1.0)
    x_out_ref[...] = x
    u = jnp.tanh(jnp.dot(x, w_ref[...], preferred_element_type=jnp.float32))
    p = jnp.dot(u, proj_ref[...], preferred_element_type=jnp.float32)

    @pl.when(i == 0)
    def _init():
        psum_ref[...] = jnp.zeros_like(psum_ref)

    psum_ref[...] = psum_ref[...] + jnp.sum(p)


def _final_graph(num, den, weight_W, weight_proj, *, blk=400):
    n = num.shape[0]
    return pl.pallas_call(
        _final_graph_body,
        grid=(n // blk,),
        in_specs=[
            pl.BlockSpec((blk, 128), lambda i: (i, 0)),
            pl.BlockSpec((blk, 1), lambda i: (i, 0)),
            pl.BlockSpec((128, 128), lambda i: (0, 0)),
            pl.BlockSpec((128, 1), lambda i: (0, 0)),
        ],
        out_specs=[
            pl.BlockSpec((blk, 128), lambda i: (i, 0)),
            pl.BlockSpec((1, 1), lambda i: (0, 0)),
        ],
        out_shape=[
            jax.ShapeDtypeStruct((n, 128), jnp.float32),
            jax.ShapeDtypeStruct((1, 1), jnp.float32),
        ],
    )(num, den, weight_W, weight_proj)


def _logits_body(g1_ref, g2_ref, a_ref, w_ref, b_ref, out_ref):
    a0 = a_ref[0]
    a1 = a_ref[1]
    of = a0 * g1_ref[...] + a1 * g2_ref[...]
    z = jnp.dot(of, w_ref[...], preferred_element_type=jnp.float32) + b_ref[...]
    m = jnp.max(z, axis=1, keepdims=True)
    z = z - m
    out_ref[...] = z - jnp.log(jnp.sum(jnp.exp(z), axis=1, keepdims=True))


def _logits(g1, g2, att, w_pad, b_pad, *, blk=512):
    b = g1.shape[0]
    return pl.pallas_call(
        _logits_body,
        grid=(b // blk,),
        in_specs=[
            pl.BlockSpec((blk, 128), lambda i: (i, 0)),
            pl.BlockSpec((blk, 128), lambda i: (i, 0)),
            pl.BlockSpec(memory_space=pltpu.SMEM),
            pl.BlockSpec((128, 128), lambda i: (0, 0)),
            pl.BlockSpec((1, 128), lambda i: (0, 0)),
        ],
        out_specs=pl.BlockSpec((blk, 128), lambda i: (i, 0)),
        out_shape=jax.ShapeDtypeStruct((b, 128), jnp.float32),
    )(g1, g2, att, w_pad, b_pad)


def _embed_mean(word_emb_pad, feat_flat, n, words, kcols, kpad):
    tb = n // L
    nb = _cdiv(tb, NW)
    cv = kcols // L

    @functools.partial(
        pl.kernel,
        out_type=jax.ShapeDtypeStruct((n, kpad), jnp.float32),
        mesh=plsc.VectorSubcoreMesh(**_MESH),
        compiler_params=pltpu.CompilerParams(needs_layout_passes=False),
        scratch_types=[
            pltpu.VMEM((L * words,), jnp.int32),
            pltpu.VMEM((L * words,), jnp.int32),
            pltpu.VMEM((2, L * words, kcols), jnp.float32),
            pltpu.VMEM((L, kpad), jnp.float32),
            pltpu.SemaphoreType.DMA,
            pltpu.SemaphoreType.DMA,
        ],
    )
    def k(we_hbm, fi_hbm, out_hbm, fidx_a, fidx_b, rows_v, obuf_v, sem_a, sem_b):
        wid = lax.axis_index("s") * NC + lax.axis_index("c")

        def jof(b):
            return jnp.minimum(wid * nb + b, tb - 1)

        def load_issue(b, buf, sem):
            j = jof(b)
            fidx = fidx_a if buf == 0 else fidx_b
            pltpu.sync_copy(fi_hbm.at[pl.ds(j * (L * words), L * words)],
                            fidx)
            return pltpu.async_copy(we_hbm.at[fidx], rows_v.at[buf], sem)

        def compute_out(b, buf):
            def node(kk, _):
                base = kk * words
                for c in range(cv):
                    a = rows_v[buf, base, pl.ds(c * L, L)]
                    for r in range(1, words):
                        a = a + rows_v[buf, base + r, pl.ds(c * L, L)]
                    obuf_v[kk, pl.ds(c * L, L)] = a * (1.0 / words)
                return 0

            lax.fori_loop(0, L, node, 0)
            pltpu.sync_copy(obuf_v, out_hbm.at[pl.ds(jof(b) * L, L)])

        load_issue(0, 0, sem_a)

        def pair(i, _):
            ba = 2 * i
            bb = ba + 1

            @pl.when(bb < nb)
            def _ib():
                load_issue(bb, 1, sem_b)

            pltpu.make_async_copy(we_hbm.at[fidx_a], rows_v.at[0],
                                  sem_a).wait()
            compute_out(ba, 0)

            @pl.when(ba + 2 < nb)
            def _ia():
                load_issue(ba + 2, 0, sem_a)

            @pl.when(bb < nb)
            def _cb():
                pltpu.make_async_copy(we_hbm.at[fidx_b], rows_v.at[1],
                                      sem_b).wait()
                compute_out(bb, 1)

            return 0

        lax.fori_loop(0, _cdiv(nb, 2), pair, 0)

    return k(word_emb_pad, feat_flat)


def _edge_aggregate(h, hsd, src, dst, n):
    e = src.shape[0]
    chunk = 2048
    n_pad = _cdiv(n, 32) * 32
    e_pad = _cdiv(e, NS * chunk) * (NS * chunk)
    eb = 48
    epc = e_pad // NS
    nq = n_pad // 4
    nqp = nq
    ndr = _cdiv(_cdiv(nq + 1, 128), L) * L
    q = _cdiv(_cdiv(nq, NS), 8) * 8
    zc = _cdiv(nqp, NS * eb)
    z = zc * eb
    dq = 8

    src_p = jnp.pad(src, (0, e_pad - e))
    dst_p = jnp.pad(dst, (0, e_pad - e), constant_values=n_pad)
    if n_pad == n:
        h_p = h
        hs = hsd[:, 0:1].reshape(n)
        hd = hsd[:, 1:2].reshape(n)
    else:
        h_p = jnp.pad(h, ((0, n_pad - n), (0, 0)))
        hs = jnp.pad(hsd[:, 0:1].reshape(n), (0, n_pad - n))
        hd = jnp.pad(hsd[:, 1:2].reshape(n), (0, n_pad - n))

    @functools.partial(
        pl.kernel,
        out_type=[
            jax.ShapeDtypeStruct((n_pad, 128), jnp.float32),
            jax.ShapeDtypeStruct((2, NC, ndr, 128), jnp.float32),
        ],
        mesh=plsc.VectorSubcoreMesh(**_MESH),
        compiler_params=pltpu.CompilerParams(needs_layout_passes=False),
        scratch_types=[
            pltpu.VMEM((n_pad,), jnp.float32),
            pltpu.VMEM((nq,), jnp.float32),
            pltpu.VMEM((ndr, 128), jnp.float32),
            pltpu.VMEM((ndr,), jnp.int32),
            pltpu.VMEM((chunk,), jnp.int32),
            pltpu.VMEM((chunk,), jnp.int32),
            pltpu.VMEM((chunk + eb,), jnp.int32),
            pltpu.VMEM((chunk + eb,), jnp.int32),
            pltpu.VMEM((2, eb), jnp.int32),
            pltpu.VMEM((2, eb), jnp.int32),
            pltpu.VMEM((2, eb), jnp.float32),
            pltpu.VMEM((2, eb, 128), jnp.float32),
            pltpu.VMEM_SHARED((nqp, 128), jnp.float32),
            pltpu.VMEM_SHARED((ndr, 128), jnp.float32),
            pltpu.SemaphoreType.DMA,
            pltpu.SemaphoreType.DMA,
            pltpu.SemaphoreType.DMA,
            pltpu.SemaphoreType.DMA,
            pltpu.SemaphoreType.DMA,
        ],
    )
    def k(h_hbm, hs_hbm, hd_hbm, src_hbm, dst_hbm, num_hbm, den_hbm,
          hs_v, hdq_v, den_v, iden_v, src_v, dst_v, sc_v, dc_v,
          gidx2_v, dloc2_v, w_v, rows_v, acc_spm, denacc_spm, sem_ga,
          sem_gb, sem_sa, sem_sb, sem_s):
        cid = lax.axis_index("c")
        sid = lax.axis_index("s")
        zero = jnp.zeros((L,), jnp.float32)
        ramp = lax.iota(jnp.int32, L)

        pltpu.sync_copy(hs_hbm, hs_v)

        for i in range(ndr // L):
            iden_v[pl.ds(i * L, L)] = ramp + (i * L)

        def phase_body(ph, _):
            base = pl.multiple_of(ph * (2 * nq) + cid * nq, 8)
            pltpu.sync_copy(hd_hbm.at[pl.ds(base, nq)], hdq_v)

            def zden(i, _):
                for c in range(8):
                    den_v[i, pl.ds(c * L, L)] = zero
                return 0

            lax.fori_loop(0, ndr, zden, 0)

            def zrow(r, _):
                for c in range(8):
                    rows_v[0, r, pl.ds(c * L, L)] = zero
                return 0

            lax.fori_loop(0, eb, zrow, 0)

            zstart = jnp.minimum(sid * z, nqp - z)

            def zacc(i, _):
                pltpu.sync_copy(rows_v.at[0], acc_spm.at[pl.ds(zstart + i * eb, eb)])
                return 0

            lax.fori_loop(0, zc, zacc, 0)

            @pl.when(sid == 0)
            def _zden_shared():
                pltpu.sync_copy(den_v, denacc_spm)

            plsc.subcore_barrier()

            def chunk_body(ci, _):
                eoff = sid * epc + ci * chunk
                pltpu.sync_copy(src_hbm.at[pl.ds(eoff, chunk)], src_v)
                pltpu.sync_copy(dst_hbm.at[pl.ds(eoff, chunk)], dst_v)

                def pre(i, _):
                    sc_v[pl.ds(i * L, L)] = ramp * 0
                    dc_v[pl.ds(i * L, L)] = ramp * 0 + nq
                    return 0

                lax.fori_loop(0, (chunk + eb) // L, pre, 0)

                def cmp_body(j, cnt):
                    s = src_v[pl.ds(j * L, L)]
                    d = dst_v[pl.ds(j * L, L)]
                    inq = (d >= base) & (d < base + nq)
                    dloc = d - base
                    csum = plsc.cumsum(inq.astype(jnp.int32))
                    idx = cnt + csum - 1
                    plsc.store_scatter(sc_v, [idx], s, mask=inq)
                    plsc.store_scatter(dc_v, [idx], dloc, mask=inq)
                    return cnt + csum[L - 1]

                cnt = lax.fori_loop(0, chunk // L, cmp_body, 0)
                nblk = lax.div(cnt + (eb - 1), eb)

                def prep(b, buf):
                    boff = b * eb
                    for j in range(eb // L):
                        s = sc_v[pl.ds(boff + j * L, L)]
                        dl = dc_v[pl.ds(boff + j * L, L)]
                        valid = dl < nq
                        dl = jnp.where(valid, dl, 0)
                        gidx2_v[buf, pl.ds(j * L, L)] = s
                        dloc2_v[buf, pl.ds(j * L, L)] = dl
                        hs16 = plsc.load_gather(hs_v, [s])
                        hd16 = plsc.load_gather(hdq_v, [dl])
                        ee = hs16 + hd16
                        w = jnp.exp(jnp.where(ee >= 0, ee, ALPHA * ee))
                        w = jnp.where(valid, w, 0.0)
                        rr = lax.shift_right_logical(dl, 7)
                        cc = lax.bitwise_and(dl, 127)
                        plsc.addupdate_scatter(den_v, [rr, cc], w)
                        w_v[buf, pl.ds(j * L, L)] = w

                def issue(buf, sem):
                    return pltpu.async_copy(h_hbm.at[gidx2_v.at[buf]],
                                            rows_v.at[buf], sem)

                def scale_scatter(buf, sem):
                    for g in range(eb // L):
                        wv = w_v[buf, pl.ds(g * L, L)]
                        for lane in range(L):
                            wr = wv[lane]
                            r = g * L + lane
                            for c in range(8):
                                rows_v[buf, r, pl.ds(c * L, L)] = (
                                    rows_v[buf, r, pl.ds(c * L, L)] * wr)
                    pltpu.async_copy(rows_v.at[buf],
                                     acc_spm.at[dloc2_v.at[buf]],
                                     sem, add=True)

                def wait_scat(buf, sem):
                    pltpu.make_async_copy(rows_v.at[buf],
                                          acc_spm.at[dloc2_v.at[buf]],
                                          sem).wait()

                @pl.when(nblk > 0)
                def _prologue():
                    prep(0, 0)
                    issue(0, sem_ga)

                def pair_body(i, _):
                    ba = 2 * i
                    bb = ba + 1

                    @pl.when(bb < nblk)
                    def _pb():
                        prep(bb, 1)

                        @pl.when(i > 0)
                        def _ws1():
                            wait_scat(1, sem_sb)

                        issue(1, sem_gb)

                    pltpu.make_async_copy(h_hbm.at[gidx2_v.at[0]],
                                          rows_v.at[0], sem_ga).wait()
                    scale_scatter(0, sem_sa)

                    @pl.when(ba + 2 < nblk)
                    def _pa():
                        prep(ba + 2, 0)
                        wait_scat(0, sem_sa)
                        issue(0, sem_ga)

                    @pl.when(bb < nblk)
                    def _cb():
                        pltpu.make_async_copy(h_hbm.at[gidx2_v.at[1]],
                                              rows_v.at[1], sem_gb).wait()
                        scale_scatter(1, sem_sb)

                    return 0

                lax.fori_loop(0, lax.shift_right_logical(nblk + 1, 1),
                              pair_body, 0)

                @pl.when(nblk > 0)
                def _dr0():
                    wait_scat(0, sem_sa)

                nlast = lax.shift_right_logical(nblk + 1, 1) * 2 - 1

                @pl.when(nblk > 1)
                def _dr1():
                    wait_scat(1, sem_sb)

                return 0

            lax.fori_loop(0, epc // chunk, chunk_body, 0)

            pltpu.async_copy(den_v, denacc_spm.at[iden_v], sem_s,
                             add=True).wait()
            plsc.subcore_barrier()

            rstart = jnp.minimum(sid * q, nq - q)
            pltpu.sync_copy(acc_spm.at[pl.ds(rstart, q)],
                            num_hbm.at[pl.ds(base + rstart, q)])
            dstart = jnp.minimum(sid * dq, ndr - dq)
            pltpu.sync_copy(denacc_spm.at[pl.ds(dstart, dq)],
                            den_hbm.at[ph, cid, pl.ds(dstart, dq)])
            plsc.subcore_barrier()
            return 0

        lax.fori_loop(0, 2, phase_body, 0)

    num_p, den_p = k(h_p, hs, hd, src_p, dst_p)
    num = num_p[:n]
    den = den_p.reshape(2 * NC, ndr * 128)[:, :nq].reshape(-1)[:n]
    return num, den.reshape(n, 1)


def _pair_gather(x1, x2, idx1, idx2):
    b = idx1.shape[0]
    r = b // NW

    @functools.partial(
        pl.kernel,
        out_type=[
            jax.ShapeDtypeStruct((b, 128), jnp.float32),
            jax.ShapeDtypeStruct((b, 128), jnp.float32),
        ],
        mesh=plsc.VectorSubcoreMesh(**_MESH),
        compiler_params=pltpu.CompilerParams(needs_layout_passes=False),
        scratch_types=[
            pltpu.VMEM((r,), jnp.int32),
            pltpu.VMEM((r, 128), jnp.float32),
            pltpu.SemaphoreType.DMA,
        ],
    )
    def k(x1_hbm, x2_hbm, i1_hbm, i2_hbm, o1_hbm, o2_hbm, idx_v, rows_v, sem):
        wid = lax.axis_index("s") * NC + lax.axis_index("c")
        base = wid * r
        pltpu.sync_copy(i1_hbm.at[pl.ds(base, r)], idx_v)
        pltpu.async_copy(x1_hbm.at[idx_v], rows_v, sem).wait()
        pltpu.sync_copy(rows_v, o1_hbm.at[pl.ds(base, r)])
        pltpu.sync_copy(i2_hbm.at[pl.ds(base, r)], idx_v)
        pltpu.async_copy(x2_hbm.at[idx_v], rows_v, sem).wait()
        pltpu.sync_copy(rows_v, o2_hbm.at[pl.ds(base, r)])

    return k(x1, x2, idx1, idx2)


def _spgat(x, den0, src, dst, W1p, a1s, a1d, W2, a2s, a2d, n, weight_W,
           weight_proj):
    h1, hsd1 = _mm_hsd(x, den0, W1p, a1s, a1d, layer2=False)
    num1, den1 = _edge_aggregate(h1, hsd1, src, dst, n)
    h2, hsd2 = _mm_hsd(num1, den1, W2, a2s, a2d, layer2=True)
    num2, den2 = _edge_aggregate(h2, hsd2, src, dst, n)
    return _final_graph(num2, den2, weight_W, weight_proj)


def kernel(feat_idx, tw_src, tw_dst, ut_src, ut_dst, tw_graph_idx, ut_graph_idx,
           word_embedding, user_embedding, tw_W1, tw_a1s, tw_a1d, tw_W2, tw_a2s,
           tw_a2d, ut_W1, ut_a1s, ut_a1d, ut_W2, ut_a2s, ut_a2d, weight_W,
           weight_proj, out_W, out_b):
    n_tw, words = feat_idx.shape
    n_ut = user_embedding.shape[0]
    nfeat = word_embedding.shape[1]
    kpad = 384
    kcols = kpad

    we_pad = jnp.pad(word_embedding, ((0, 0), (0, kcols - nfeat)))
    feat_flat = feat_idx.reshape(-1).astype(jnp.int32)
    twt_Xp = _embed_mean(we_pad, feat_flat, n_tw, words, kcols, kpad)
    ue_p = jnp.pad(user_embedding, ((0, 0), (0, kpad - nfeat)))
    tw_W1p = jnp.pad(tw_W1, ((0, kpad - nfeat), (0, 0)))
    ut_W1p = jnp.pad(ut_W1, ((0, kpad - nfeat), (0, 0)))

    one_tw = jnp.ones((n_tw, 1), jnp.float32)
    one_ut = jnp.ones((n_ut, 1), jnp.float32)
    tw_X, tw_psum = _spgat(twt_Xp, one_tw, tw_src.astype(jnp.int32),
                           tw_dst.astype(jnp.int32), tw_W1p, tw_a1s, tw_a1d,
                           tw_W2, tw_a2s, tw_a2d, n_tw, weight_W, weight_proj)
    tu_X, tu_psum = _spgat(ue_p, one_ut, ut_src.astype(jnp.int32),
                           ut_dst.astype(jnp.int32), ut_W1p, ut_a1s, ut_a1d,
                           ut_W2, ut_a2s, ut_a2d, n_ut, weight_W, weight_proj)

    att_tw = tw_psum[0, 0] / n_tw
    att_tu = tu_psum[0, 0] / n_ut
    m = jnp.maximum(att_tw, att_tu)
    e0 = jnp.exp(att_tw - m)
    e1 = jnp.exp(att_tu - m)
    att = jnp.stack([e0, e1]) / (e0 + e1)

    g1, g2 = _pair_gather(tw_X, tu_X, tw_graph_idx.astype(jnp.int32),
                          ut_graph_idx.astype(jnp.int32))

    nclass = out_W.shape[0]
    w_pad = jnp.pad(out_W.T, ((0, 0), (0, 128 - nclass)))
    b_pad = jnp.pad(out_b[None, :], ((0, 0), (0, 128 - nclass)),
                    constant_values=NEG)
    lp = _logits(g1, g2, att, w_pad, b_pad)
    return lp[:, :nclass]


# speedup vs baseline: 1.0330x; 1.0315x over previous
import functools

import jax
import jax.numpy as jnp
from jax import lax
from jax.experimental import pallas as pl
from jax.experimental.pallas import tpu as pltpu
from jax.experimental.pallas import tpu_sc as plsc

ALPHA = 0.2
EPS = 1e-16
NEG = -1e9

NC = 2
NS = 16
NW = NC * NS
L = 16

_MESH = dict(core_axis_name="c", subcore_axis_name="s", num_cores=NC,
             num_subcores=NS)


def _cdiv(a, b):
    return (a + b - 1) // b


def _mm_hsd_body(num_ref, den_ref, w_ref, asd_ref, h_ref, hsd_ref, *, layer2):
    x = num_ref[...]
    if layer2:
        x = x / (den_ref[...] + EPS)
        x = jnp.where(x > 0, x, jnp.exp(x) - 1.0)
    h = jnp.dot(x, w_ref[...], preferred_element_type=jnp.float32)
    h_ref[...] = h
    hsd_ref[...] = jnp.dot(h, asd_ref[...], preferred_element_type=jnp.float32)


def _mm_hsd(x, den, W, a_s, a_d, *, layer2, blk=400):
    n, k = x.shape
    asd = jnp.stack([a_s, a_d], axis=1)
    kfn = functools.partial(_mm_hsd_body, layer2=layer2)
    return pl.pallas_call(
        kfn,
        grid=(n // blk,),
        in_specs=[
            pl.BlockSpec((blk, k), lambda i: (i, 0)),
            pl.BlockSpec((blk, 1), lambda i: (i, 0)),
            pl.BlockSpec((k, 128), lambda i: (0, 0)),
            pl.BlockSpec((128, 2), lambda i: (0, 0)),
        ],
        out_specs=[
            pl.BlockSpec((blk, 128), lambda i: (i, 0)),
            pl.BlockSpec((blk, 2), lambda i: (i, 0)),
        ],
        out_shape=[
            jax.ShapeDtypeStruct((n, 128), jnp.float32),
            jax.ShapeDtypeStruct((n, 2), jnp.float32),
        ],
    )(x, den, W, asd)


def _final_graph_body(num_ref, den_ref, w_ref, proj_ref, x_out_ref, psum_ref):
    i = pl.program_id(0)
    x = num_ref[...] / (den_ref[...] + EPS)
    x = jnp.where(x > 0, x, jnp.exp(x) - 1.0)
    x_out_ref[...] = x
    u = jnp.tanh(jnp.dot(x, w_ref[...], preferred_element_type=jnp.float32))
    p = jnp.dot(u, proj_ref[...], preferred_element_type=jnp.float32)

    @pl.when(i == 0)
    def _init():
        psum_ref[...] = jnp.zeros_like(psum_ref)

    psum_ref[...] = psum_ref[...] + jnp.sum(p)


def _final_graph(num, den, weight_W, weight_proj, *, blk=400):
    n = num.shape[0]
    return pl.pallas_call(
        _final_graph_body,
        grid=(n // blk,),
        in_specs=[
            pl.BlockSpec((blk, 128), lambda i: (i, 0)),
            pl.BlockSpec((blk, 1), lambda i: (i, 0)),
            pl.BlockSpec((128, 128), lambda i: (0, 0)),
            pl.BlockSpec((128, 1), lambda i: (0, 0)),
        ],
        out_specs=[
            pl.BlockSpec((blk, 128), lambda i: (i, 0)),
            pl.BlockSpec((1, 1), lambda i: (0, 0)),
        ],
        out_shape=[
            jax.ShapeDtypeStruct((n, 128), jnp.float32),
            jax.ShapeDtypeStruct((1, 1), jnp.float32),
        ],
    )(num, den, weight_W, weight_proj)


def _logits_body(g1_ref, g2_ref, a_ref, w_ref, b_ref, out_ref):
    a0 = a_ref[0]
    a1 = a_ref[1]
    of = a0 * g1_ref[...] + a1 * g2_ref[...]
    z = jnp.dot(of, w_ref[...], preferred_element_type=jnp.float32) + b_ref[...]
    m = jnp.max(z, axis=1, keepdims=True)
    z = z - m
    out_ref[...] = z - jnp.log(jnp.sum(jnp.exp(z), axis=1, keepdims=True))


def _logits(g1, g2, att, w_pad, b_pad, *, blk=512):
    b = g1.shape[0]
    return pl.pallas_call(
        _logits_body,
        grid=(b // blk,),
        in_specs=[
            pl.BlockSpec((blk, 128), lambda i: (i, 0)),
            pl.BlockSpec((blk, 128), lambda i: (i, 0)),
            pl.BlockSpec(memory_space=pltpu.SMEM),
            pl.BlockSpec((128, 128), lambda i: (0, 0)),
            pl.BlockSpec((1, 128), lambda i: (0, 0)),
        ],
        out_specs=pl.BlockSpec((blk, 128), lambda i: (i, 0)),
        out_shape=jax.ShapeDtypeStruct((b, 128), jnp.float32),
    )(g1, g2, att, w_pad, b_pad)


def _embed_mean(word_emb, feat_flat, n, words):
    kc = word_emb.shape[1]
    tb = n // L
    nb = _cdiv(tb, NW)
    cols = list(range(0, kc - L + 1, L))
    if cols[-1] != kc - L:
        cols.append(kc - L)

    @functools.partial(
        pl.kernel,
        out_type=jax.ShapeDtypeStruct((n, kc), jnp.float32),
        mesh=plsc.VectorSubcoreMesh(**_MESH),
        compiler_params=pltpu.CompilerParams(needs_layout_passes=False,
                                             use_tc_tiling_on_sc=False),
        scratch_types=[
            pltpu.VMEM((L * words,), jnp.int32),
            pltpu.VMEM((L * words,), jnp.int32),
            pltpu.VMEM((2, L * words, kc), jnp.float32),
            pltpu.VMEM((L, kc), jnp.float32),
            pltpu.SemaphoreType.DMA,
            pltpu.SemaphoreType.DMA,
        ],
    )
    def k(we_hbm, fi_hbm, out_hbm, fidx_a, fidx_b, rows_v, obuf_v, sem_a,
          sem_b):
        wid = lax.axis_index("s") * NC + lax.axis_index("c")

        def jof(b):
            return jnp.minimum(wid * nb + b, tb - 1)

        def load_issue(b, buf, sem):
            j = jof(b)
            fidx = fidx_a if buf == 0 else fidx_b
            pltpu.sync_copy(fi_hbm.at[pl.ds(j * (L * words), L * words)],
                            fidx)
            return pltpu.async_copy(we_hbm.at[fidx], rows_v.at[buf], sem)

        def compute_out(b, buf):
            def node(kk, _):
                base = kk * words
                for c in cols:
                    a = rows_v[buf, base, pl.ds(c, L)]
                    for r in range(1, words):
                        a = a + rows_v[buf, base + r, pl.ds(c, L)]
                    obuf_v[kk, pl.ds(c, L)] = a * (1.0 / words)
                return 0

            lax.fori_loop(0, L, node, 0)
            pltpu.sync_copy(obuf_v, out_hbm.at[pl.ds(jof(b) * L, L)])

        load_issue(0, 0, sem_a)

        def pair(i, _):
            ba = 2 * i
            bb = ba + 1

            @pl.when(bb < nb)
            def _ib():
                load_issue(bb, 1, sem_b)

            pltpu.make_async_copy(we_hbm.at[fidx_a], rows_v.at[0],
                                  sem_a).wait()
            compute_out(ba, 0)

            @pl.when(ba + 2 < nb)
            def _ia():
                load_issue(ba + 2, 0, sem_a)

            @pl.when(bb < nb)
            def _cb():
                pltpu.make_async_copy(we_hbm.at[fidx_b], rows_v.at[1],
                                      sem_b).wait()
                compute_out(bb, 1)

            return 0

        lax.fori_loop(0, _cdiv(nb, 2), pair, 0)

    return k(word_emb, feat_flat)


def _edge_aggregate(h, hsd, src, dst, n):
    e = src.shape[0]
    chunk = 2048
    n_pad = _cdiv(n, 32) * 32
    e_pad = _cdiv(e, NS * chunk) * (NS * chunk)
    eb = 48
    epc = e_pad // NS
    nq = n_pad // 4
    nqp = nq
    ndr = _cdiv(_cdiv(nq + 1, 128), L) * L
    q = _cdiv(_cdiv(nq, NS), 8) * 8
    zc = _cdiv(nqp, NS * eb)
    z = zc * eb
    dq = 8

    src_p = jnp.pad(src, (0, e_pad - e))
    dst_p = jnp.pad(dst, (0, e_pad - e), constant_values=n_pad)
    if n_pad == n:
        h_p = h
        hs = hsd[:, 0:1].reshape(n)
        hd = hsd[:, 1:2].reshape(n)
    else:
        h_p = jnp.pad(h, ((0, n_pad - n), (0, 0)))
        hs = jnp.pad(hsd[:, 0:1].reshape(n), (0, n_pad - n))
        hd = jnp.pad(hsd[:, 1:2].reshape(n), (0, n_pad - n))

    @functools.partial(
        pl.kernel,
        out_type=[
            jax.ShapeDtypeStruct((n_pad, 128), jnp.float32),
            jax.ShapeDtypeStruct((2, NC, ndr, 128), jnp.float32),
        ],
        mesh=plsc.VectorSubcoreMesh(**_MESH),
        compiler_params=pltpu.CompilerParams(needs_layout_passes=False),
        scratch_types=[
            pltpu.VMEM((n_pad,), jnp.float32),
            pltpu.VMEM((nq,), jnp.float32),
            pltpu.VMEM((ndr, 128), jnp.float32),
            pltpu.VMEM((ndr,), jnp.int32),
            pltpu.VMEM((chunk,), jnp.int32),
            pltpu.VMEM((chunk,), jnp.int32),
            pltpu.VMEM((chunk + eb,), jnp.int32),
            pltpu.VMEM((chunk + eb,), jnp.int32),
            pltpu.VMEM((2, eb), jnp.int32),
            pltpu.VMEM((2, eb), jnp.int32),
            pltpu.VMEM((2, eb), jnp.float32),
            pltpu.VMEM((2, eb, 128), jnp.float32),
            pltpu.VMEM_SHARED((nqp, 128), jnp.float32),
            pltpu.VMEM_SHARED((ndr, 128), jnp.float32),
            pltpu.SemaphoreType.DMA,
            pltpu.SemaphoreType.DMA,
            pltpu.SemaphoreType.DMA,
            pltpu.SemaphoreType.DMA,
            pltpu.SemaphoreType.DMA,
        ],
    )
    def k(h_hbm, hs_hbm, hd_hbm, src_hbm, dst_hbm, num_hbm, den_hbm,
          hs_v, hdq_v, den_v, iden_v, src_v, dst_v, sc_v, dc_v,
          gidx2_v, dloc2_v, w_v, rows_v, acc_spm, denacc_spm, sem_ga,
          sem_gb, sem_sa, sem_sb, sem_s):
        cid = lax.axis_index("c")
        sid = lax.axis_index("s")
        zero = jnp.zeros((L,), jnp.float32)
        ramp = lax.iota(jnp.int32, L)

        pltpu.sync_copy(hs_hbm, hs_v)

        for i in range(ndr // L):
            iden_v[pl.ds(i * L, L)] = ramp + (i * L)

        def phase_body(ph, _):
            base = pl.multiple_of(ph * (2 * nq) + cid * nq, 8)
            pltpu.sync_copy(hd_hbm.at[pl.ds(base, nq)], hdq_v)

            def zden(i, _):
                for c in range(8):
                    den_v[i, pl.ds(c * L, L)] = zero
                return 0

            lax.fori_loop(0, ndr, zden, 0)

            def zrow(r, _):
                for c in range(8):
                    rows_v[0, r, pl.ds(c * L, L)] = zero
                return 0

            lax.fori_loop(0, eb, zrow, 0)

            zstart = jnp.minimum(sid * z, nqp - z)

            def zacc(i, _):
                pltpu.sync_copy(rows_v.at[0], acc_spm.at[pl.ds(zstart + i * eb, eb)])
                return 0

            lax.fori_loop(0, zc, zacc, 0)

            @pl.when(sid == 0)
            def _zden_shared():
                pltpu.sync_copy(den_v, denacc_spm)

            plsc.subcore_barrier()

            def chunk_body(ci, _):
                eoff = sid * epc + ci * chunk
                pltpu.sync_copy(src_hbm.at[pl.ds(eoff, chunk)], src_v)
                pltpu.sync_copy(dst_hbm.at[pl.ds(eoff, chunk)], dst_v)

                def pre(i, _):
                    sc_v[pl.ds(i * L, L)] = ramp * 0
                    dc_v[pl.ds(i * L, L)] = ramp * 0 + nq
                    return 0

                lax.fori_loop(0, (chunk + eb) // L, pre, 0)

                def cmp_body(j, cnt):
                    s = src_v[pl.ds(j * L, L)]
                    d = dst_v[pl.ds(j * L, L)]
                    inq = (d >= base) & (d < base + nq)
                    dloc = d - base
                    csum = plsc.cumsum(inq.astype(jnp.int32))
                    idx = cnt + csum - 1
                    plsc.store_scatter(sc_v, [idx], s, mask=inq)
                    plsc.store_scatter(dc_v, [idx], dloc, mask=inq)
                    return cnt + csum[L - 1]

                cnt = lax.fori_loop(0, chunk // L, cmp_body, 0)
                nblk = lax.div(cnt + (eb - 1), eb)

                def prep(b, buf):
                    boff = b * eb
                    for j in range(eb // L):
                        s = sc_v[pl.ds(boff + j * L, L)]
                        dl = dc_v[pl.ds(boff + j * L, L)]
                        valid = dl < nq
                        dl = jnp.where(valid, dl, 0)
                        gidx2_v[buf, pl.ds(j * L, L)] = s
                        dloc2_v[buf, pl.ds(j * L, L)] = dl
                        hs16 = plsc.load_gather(hs_v, [s])
                        hd16 = plsc.load_gather(hdq_v, [dl])
                        ee = hs16 + hd16
                        w = jnp.exp(jnp.where(ee >= 0, ee, ALPHA * ee))
                        w = jnp.where(valid, w, 0.0)
                        rr = lax.shift_right_logical(dl, 7)
                        cc = lax.bitwise_and(dl, 127)
                        plsc.addupdate_scatter(den_v, [rr, cc], w)
                        w_v[buf, pl.ds(j * L, L)] = w

                def issue(buf, sem):
                    return pltpu.async_copy(h_hbm.at[gidx2_v.at[buf]],
                                            rows_v.at[buf], sem)

                def scale_scatter(buf, sem):
                    for g in range(eb // L):
                        wv = w_v[buf, pl.ds(g * L, L)]
                        for lane in range(L):
                            wr = wv[lane]
                            r = g * L + lane
                            for c in range(8):
                                rows_v[buf, r, pl.ds(c * L, L)] = (
                                    rows_v[buf, r, pl.ds(c * L, L)] * wr)
                    pltpu.async_copy(rows_v.at[buf],
                                     acc_spm.at[dloc2_v.at[buf]],
                                     sem, add=True)

                def wait_scat(buf, sem):
                    pltpu.make_async_copy(rows_v.at[buf],
                                          acc_spm.at[dloc2_v.at[buf]],
                                          sem).wait()

                @pl.when(nblk > 0)
                def _prologue():
                    prep(0, 0)
                    issue(0, sem_ga)

                def pair_body(i, _):
                    ba = 2 * i
                    bb = ba + 1

                    @pl.when(bb < nblk)
                    def _pb():
                        prep(bb, 1)

                        @pl.when(i > 0)
                        def _ws1():
                            wait_scat(1, sem_sb)

                        issue(1, sem_gb)

                    pltpu.make_async_copy(h_hbm.at[gidx2_v.at[0]],
                                          rows_v.at[0], sem_ga).wait()
                    scale_scatter(0, sem_sa)

                    @pl.when(ba + 2 < nblk)
                    def _pa():
                        prep(ba + 2, 0)
                        wait_scat(0, sem_sa)
                        issue(0, sem_ga)

                    @pl.when(bb < nblk)
                    def _cb():
                        pltpu.make_async_copy(h_hbm.at[gidx2_v.at[1]],
                                              rows_v.at[1], sem_gb).wait()
                        scale_scatter(1, sem_sb)

                    return 0

                lax.fori_loop(0, lax.shift_right_logical(nblk + 1, 1),
                              pair_body, 0)

                @pl.when(nblk > 0)
                def _dr0():
                    wait_scat(0, sem_sa)

                @pl.when(nblk > 1)
                def _dr1():
                    wait_scat(1, sem_sb)

                return 0

            lax.fori_loop(0, epc // chunk, chunk_body, 0)

            pltpu.async_copy(den_v, denacc_spm.at[iden_v], sem_s,
                             add=True).wait()
            plsc.subcore_barrier()

            rstart = jnp.minimum(sid * q, nq - q)
            pltpu.sync_copy(acc_spm.at[pl.ds(rstart, q)],
                            num_hbm.at[pl.ds(base + rstart, q)])
            dstart = jnp.minimum(sid * dq, ndr - dq)
            pltpu.sync_copy(denacc_spm.at[pl.ds(dstart, dq)],
                            den_hbm.at[ph, cid, pl.ds(dstart, dq)])
            plsc.subcore_barrier()
            return 0

        lax.fori_loop(0, 2, phase_body, 0)

    num_p, den_p = k(h_p, hs, hd, src_p, dst_p)
    num = num_p[:n]
    den = den_p.reshape(2 * NC, ndr * 128)[:, :nq].reshape(-1)[:n]
    return num, den.reshape(n, 1)


def _pair_gather(x1, x2, idx1, idx2):
    b = idx1.shape[0]
    r = b // NW

    @functools.partial(
        pl.kernel,
        out_type=[
            jax.ShapeDtypeStruct((b, 128), jnp.float32),
            jax.ShapeDtypeStruct((b, 128), jnp.float32),
        ],
        mesh=plsc.VectorSubcoreMesh(**_MESH),
        compiler_params=pltpu.CompilerParams(needs_layout_passes=False),
        scratch_types=[
            pltpu.VMEM((r,), jnp.int32),
            pltpu.VMEM((r, 128), jnp.float32),
            pltpu.SemaphoreType.DMA,
        ],
    )
    def k(x1_hbm, x2_hbm, i1_hbm, i2_hbm, o1_hbm, o2_hbm, idx_v, rows_v, sem):
        wid = lax.axis_index("s") * NC + lax.axis_index("c")
        base = wid * r
        pltpu.sync_copy(i1_hbm.at[pl.ds(base, r)], idx_v)
        pltpu.async_copy(x1_hbm.at[idx_v], rows_v, sem).wait()
        pltpu.sync_copy(rows_v, o1_hbm.at[pl.ds(base, r)])
        pltpu.sync_copy(i2_hbm.at[pl.ds(base, r)], idx_v)
        pltpu.async_copy(x2_hbm.at[idx_v], rows_v, sem).wait()
        pltpu.sync_copy(rows_v, o2_hbm.at[pl.ds(base, r)])

    return k(x1, x2, idx1, idx2)


def _spgat(x, den0, src, dst, W1p, a1s, a1d, W2, a2s, a2d, n, weight_W,
           weight_proj):
    h1, hsd1 = _mm_hsd(x, den0, W1p, a1s, a1d, layer2=False)
    num1, den1 = _edge_aggregate(h1, hsd1, src, dst, n)
    h2, hsd2 = _mm_hsd(num1, den1, W2, a2s, a2d, layer2=True)
    num2, den2 = _edge_aggregate(h2, hsd2, src, dst, n)
    return _final_graph(num2, den2, weight_W, weight_proj)


def kernel(feat_idx, tw_src, tw_dst, ut_src, ut_dst, tw_graph_idx, ut_graph_idx,
           word_embedding, user_embedding, tw_W1, tw_a1s, tw_a1d, tw_W2, tw_a2s,
           tw_a2d, ut_W1, ut_a1s, ut_a1d, ut_W2, ut_a2s, ut_a2d, weight_W,
           weight_proj, out_W, out_b):
    n_tw, words = feat_idx.shape
    n_ut = user_embedding.shape[0]

    feat_flat = feat_idx.reshape(-1).astype(jnp.int32)
    twt_X = _embed_mean(word_embedding, feat_flat, n_tw, words)

    one_tw = jnp.ones((n_tw, 1), jnp.float32)
    one_ut = jnp.ones((n_ut, 1), jnp.float32)
    tw_X, tw_psum = _spgat(twt_X, one_tw, tw_src.astype(jnp.int32),
                           tw_dst.astype(jnp.int32), tw_W1, tw_a1s, tw_a1d,
                           tw_W2, tw_a2s, tw_a2d, n_tw, weight_W, weight_proj)
    tu_X, tu_psum = _spgat(user_embedding, one_ut, ut_src.astype(jnp.int32),
                           ut_dst.astype(jnp.int32), ut_W1, ut_a1s, ut_a1d,
                           ut_W2, ut_a2s, ut_a2d, n_ut, weight_W, weight_proj)

    att_tw = tw_psum[0, 0] / n_tw
    att_tu = tu_psum[0, 0] / n_ut
    m = jnp.maximum(att_tw, att_tu)
    e0 = jnp.exp(att_tw - m)
    e1 = jnp.exp(att_tu - m)
    att = jnp.stack([e0, e1]) / (e0 + e1)

    g1, g2 = _pair_gather(tw_X, tu_X, tw_graph_idx.astype(jnp.int32),
                          ut_graph_idx.astype(jnp.int32))

    nclass = out_W.shape[0]
    w_pad = jnp.pad(out_W.T, ((0, 0), (0, 128 - nclass)))
    b_pad = jnp.pad(out_b[None, :], ((0, 0), (0, 128 - nclass)),
                    constant_values=NEG)
    lp = _logits(g1, g2, att, w_pad, b_pad)
    return lp[:, :nclass]


# speedup vs baseline: 1.2265x; 1.1873x over previous
import functools

import jax
import jax.numpy as jnp
from jax import lax
from jax.experimental import pallas as pl
from jax.experimental.pallas import tpu as pltpu
from jax.experimental.pallas import tpu_sc as plsc

ALPHA = 0.2
EPS = 1e-16
NEG = -1e9

NC = 2
NS = 16
NW = NC * NS
L = 16

_MESH = dict(core_axis_name="c", subcore_axis_name="s", num_cores=NC,
             num_subcores=NS)


def _cdiv(a, b):
    return (a + b - 1) // b


def _mm_hsd_body(num_ref, den_ref, w_ref, asd_ref, h_ref, hsd_ref, *, layer2):
    x = num_ref[...]
    if layer2:
        x = x / (den_ref[...] + EPS)
        x = jnp.where(x > 0, x, jnp.exp(x) - 1.0)
    h = jnp.dot(x, w_ref[...], preferred_element_type=jnp.float32)
    h_ref[...] = h
    hsd_ref[...] = jnp.dot(h, asd_ref[...], preferred_element_type=jnp.float32)


def _mm_hsd(x, den, W, a_s, a_d, *, layer2, blk=400):
    n, k = x.shape
    asd = jnp.stack([a_s, a_d], axis=1)
    kfn = functools.partial(_mm_hsd_body, layer2=layer2)
    return pl.pallas_call(
        kfn,
        grid=(n // blk,),
        in_specs=[
            pl.BlockSpec((blk, k), lambda i: (i, 0)),
            pl.BlockSpec((blk, 1), lambda i: (i, 0)),
            pl.BlockSpec((k, 128), lambda i: (0, 0)),
            pl.BlockSpec((128, 2), lambda i: (0, 0)),
        ],
        out_specs=[
            pl.BlockSpec((blk, 128), lambda i: (i, 0)),
            pl.BlockSpec((blk, 2), lambda i: (i, 0)),
        ],
        out_shape=[
            jax.ShapeDtypeStruct((n, 128), jnp.float32),
            jax.ShapeDtypeStruct((n, 2), jnp.float32),
        ],
    )(x, den, W, asd)


def _final_graph_body(num_ref, den_ref, w_ref, proj_ref, x_out_ref, psum_ref):
    i = pl.program_id(0)
    x = num_ref[...] / (den_ref[...] + EPS)
    x = jnp.where(x > 0, x, jnp.exp(x) - 1.0)
    x_out_ref[...] = x
    u = jnp.tanh(jnp.dot(x, w_ref[...], preferred_element_type=jnp.float32))
    p = jnp.dot(u, proj_ref[...], preferred_element_type=jnp.float32)

    @pl.when(i == 0)
    def _init():
        psum_ref[...] = jnp.zeros_like(psum_ref)

    psum_ref[...] = psum_ref[...] + jnp.sum(p)


def _final_graph(num, den, weight_W, weight_proj, *, blk=400):
    n = num.shape[0]
    return pl.pallas_call(
        _final_graph_body,
        grid=(n // blk,),
        in_specs=[
            pl.BlockSpec((blk, 128), lambda i: (i, 0)),
            pl.BlockSpec((blk, 1), lambda i: (i, 0)),
            pl.BlockSpec((128, 128), lambda i: (0, 0)),
            pl.BlockSpec((128, 1), lambda i: (0, 0)),
        ],
        out_specs=[
            pl.BlockSpec((blk, 128), lambda i: (i, 0)),
            pl.BlockSpec((1, 1), lambda i: (0, 0)),
        ],
        out_shape=[
            jax.ShapeDtypeStruct((n, 128), jnp.float32),
            jax.ShapeDtypeStruct((1, 1), jnp.float32),
        ],
    )(num, den, weight_W, weight_proj)


def _logits_body(g1_ref, g2_ref, a_ref, w_ref, b_ref, out_ref):
    a0 = a_ref[0]
    a1 = a_ref[1]
    of = a0 * g1_ref[...] + a1 * g2_ref[...]
    z = jnp.dot(of, w_ref[...], preferred_element_type=jnp.float32) + b_ref[...]
    m = jnp.max(z, axis=1, keepdims=True)
    z = z - m
    out_ref[...] = z - jnp.log(jnp.sum(jnp.exp(z), axis=1, keepdims=True))


def _logits(g1, g2, att, w_pad, b_pad, *, blk=512):
    b = g1.shape[0]
    return pl.pallas_call(
        _logits_body,
        grid=(b // blk,),
        in_specs=[
            pl.BlockSpec((blk, 128), lambda i: (i, 0)),
            pl.BlockSpec((blk, 128), lambda i: (i, 0)),
            pl.BlockSpec(memory_space=pltpu.SMEM),
            pl.BlockSpec((128, 128), lambda i: (0, 0)),
            pl.BlockSpec((1, 128), lambda i: (0, 0)),
        ],
        out_specs=pl.BlockSpec((blk, 128), lambda i: (i, 0)),
        out_shape=jax.ShapeDtypeStruct((b, 128), jnp.float32),
    )(g1, g2, att, w_pad, b_pad)


def _embed_mean(word_emb, feat_flat, n, words):
    kc = word_emb.shape[1]
    tb = n // L
    nb = _cdiv(tb, NW)
    cols = list(range(0, kc - L + 1, L))
    if cols[-1] != kc - L:
        cols.append(kc - L)

    @functools.partial(
        pl.kernel,
        out_type=jax.ShapeDtypeStruct((n, kc), jnp.float32),
        mesh=plsc.VectorSubcoreMesh(**_MESH),
        compiler_params=pltpu.CompilerParams(needs_layout_passes=False,
                                             use_tc_tiling_on_sc=False),
        scratch_types=[
            pltpu.VMEM((L * words,), jnp.int32),
            pltpu.VMEM((L * words,), jnp.int32),
            pltpu.VMEM((2, L * words, kc), jnp.float32),
            pltpu.VMEM((L, kc), jnp.float32),
            pltpu.SemaphoreType.DMA,
            pltpu.SemaphoreType.DMA,
        ],
    )
    def k(we_hbm, fi_hbm, out_hbm, fidx_a, fidx_b, rows_v, obuf_v, sem_a,
          sem_b):
        wid = lax.axis_index("s") * NC + lax.axis_index("c")

        def jof(b):
            return jnp.minimum(wid * nb + b, tb - 1)

        def load_issue(b, buf, sem):
            j = jof(b)
            fidx = fidx_a if buf == 0 else fidx_b
            pltpu.sync_copy(fi_hbm.at[pl.ds(j * (L * words), L * words)],
                            fidx)
            return pltpu.async_copy(we_hbm.at[fidx], rows_v.at[buf], sem)

        def compute_out(b, buf):
            def node(kk, _):
                base = kk * words
                for c in cols:
                    a = rows_v[buf, base, pl.ds(c, L)]
                    for r in range(1, words):
                        a = a + rows_v[buf, base + r, pl.ds(c, L)]
                    obuf_v[kk, pl.ds(c, L)] = a * (1.0 / words)
                return 0

            lax.fori_loop(0, L, node, 0)
            pltpu.sync_copy(obuf_v, out_hbm.at[pl.ds(jof(b) * L, L)])

        load_issue(0, 0, sem_a)

        def pair(i, _):
            ba = 2 * i
            bb = ba + 1

            @pl.when(bb < nb)
            def _ib():
                load_issue(bb, 1, sem_b)

            pltpu.make_async_copy(we_hbm.at[fidx_a], rows_v.at[0],
                                  sem_a).wait()
            compute_out(ba, 0)

            @pl.when(ba + 2 < nb)
            def _ia():
                load_issue(ba + 2, 0, sem_a)

            @pl.when(bb < nb)
            def _cb():
                pltpu.make_async_copy(we_hbm.at[fidx_b], rows_v.at[1],
                                      sem_b).wait()
                compute_out(bb, 1)

            return 0

        lax.fori_loop(0, _cdiv(nb, 2), pair, 0)

    return k(word_emb, feat_flat)


def _edge_aggregate(h, hsd, src, dst, n):
    e = src.shape[0]
    chunk = 2048
    n_pad = _cdiv(n, 32) * 32
    e_pad = _cdiv(e, NS * chunk) * (NS * chunk)
    eb = 32
    epc = e_pad // NS
    nq = n_pad // 4
    nqp = nq
    ndr = _cdiv(_cdiv(nq + 1, 128), L) * L
    q = _cdiv(_cdiv(nq, NS), 8) * 8
    zc = _cdiv(nqp, NS * eb)
    z = zc * eb
    dq = 8

    src_p = jnp.pad(src, (0, e_pad - e))
    dst_p = jnp.pad(dst, (0, e_pad - e), constant_values=n_pad)
    if n_pad == n:
        h_p = h
        hs = hsd[:, 0:1].reshape(n)
        hd = hsd[:, 1:2].reshape(n)
    else:
        h_p = jnp.pad(h, ((0, n_pad - n), (0, 0)))
        hs = jnp.pad(hsd[:, 0:1].reshape(n), (0, n_pad - n))
        hd = jnp.pad(hsd[:, 1:2].reshape(n), (0, n_pad - n))

    @functools.partial(
        pl.kernel,
        out_type=[
            jax.ShapeDtypeStruct((n_pad, 128), jnp.float32),
            jax.ShapeDtypeStruct((2, NC, ndr, 128), jnp.float32),
        ],
        mesh=plsc.VectorSubcoreMesh(**_MESH),
        compiler_params=pltpu.CompilerParams(needs_layout_passes=False),
        scratch_types=[
            pltpu.VMEM((n_pad,), jnp.float32),
            pltpu.VMEM((nq,), jnp.float32),
            pltpu.VMEM((ndr, 128), jnp.float32),
            pltpu.VMEM((ndr,), jnp.int32),
            pltpu.VMEM((chunk,), jnp.int32),
            pltpu.VMEM((chunk,), jnp.int32),
            pltpu.VMEM((chunk + eb,), jnp.int32),
            pltpu.VMEM((chunk + eb,), jnp.int32),
            pltpu.VMEM((3, eb), jnp.int32),
            pltpu.VMEM((3, eb), jnp.int32),
            pltpu.VMEM((3, eb), jnp.float32),
            pltpu.VMEM((3, eb, 128), jnp.float32),
            pltpu.VMEM_SHARED((nqp, 128), jnp.float32),
            pltpu.VMEM_SHARED((ndr, 128), jnp.float32),
            [pltpu.SemaphoreType.DMA] * 3,
            [pltpu.SemaphoreType.DMA] * 3,
            pltpu.SemaphoreType.DMA,
        ],
    )
    def k(h_hbm, hs_hbm, hd_hbm, src_hbm, dst_hbm, num_hbm, den_hbm,
          hs_v, hdq_v, den_v, iden_v, src_v, dst_v, sc_v, dc_v,
          gidx2_v, dloc2_v, w_v, rows_v, acc_spm, denacc_spm, sem_g,
          sem_sc, sem_s):
        cid = lax.axis_index("c")
        sid = lax.axis_index("s")
        zero = jnp.zeros((L,), jnp.float32)
        ramp = lax.iota(jnp.int32, L)

        pltpu.sync_copy(hs_hbm, hs_v)

        for i in range(ndr // L):
            iden_v[pl.ds(i * L, L)] = ramp + (i * L)

        def phase_body(ph, _):
            base = pl.multiple_of(ph * (2 * nq) + cid * nq, 8)
            pltpu.sync_copy(hd_hbm.at[pl.ds(base, nq)], hdq_v)

            def zden(i, _):
                for c in range(8):
                    den_v[i, pl.ds(c * L, L)] = zero
                return 0

            lax.fori_loop(0, ndr, zden, 0)

            def zrow(r, _):
                for c in range(8):
                    rows_v[0, r, pl.ds(c * L, L)] = zero
                return 0

            lax.fori_loop(0, eb, zrow, 0)

            zstart = jnp.minimum(sid * z, nqp - z)

            def zacc(i, _):
                pltpu.sync_copy(rows_v.at[0], acc_spm.at[pl.ds(zstart + i * eb, eb)])
                return 0

            lax.fori_loop(0, zc, zacc, 0)

            @pl.when(sid == 0)
            def _zden_shared():
                pltpu.sync_copy(den_v, denacc_spm)

            plsc.subcore_barrier()

            def chunk_body(ci, _):
                eoff = sid * epc + ci * chunk
                pltpu.sync_copy(src_hbm.at[pl.ds(eoff, chunk)], src_v)
                pltpu.sync_copy(dst_hbm.at[pl.ds(eoff, chunk)], dst_v)

                def pre(i, _):
                    sc_v[pl.ds(i * L, L)] = ramp * 0
                    dc_v[pl.ds(i * L, L)] = ramp * 0 + nq
                    return 0

                lax.fori_loop(0, (chunk + eb) // L, pre, 0)

                def cmp_body(j, cnt):
                    s = src_v[pl.ds(j * L, L)]
                    d = dst_v[pl.ds(j * L, L)]
                    inq = (d >= base) & (d < base + nq)
                    dloc = d - base
                    csum = plsc.cumsum(inq.astype(jnp.int32))
                    idx = cnt + csum - 1
                    plsc.store_scatter(sc_v, [idx], s, mask=inq)
                    plsc.store_scatter(dc_v, [idx], dloc, mask=inq)
                    return cnt + csum[L - 1]

                cnt = lax.fori_loop(0, chunk // L, cmp_body, 0)
                nblk = lax.div(cnt + (eb - 1), eb)

                def prep(b, buf):
                    boff = b * eb
                    for j in range(eb // L):
                        s = sc_v[pl.ds(boff + j * L, L)]
                        dl = dc_v[pl.ds(boff + j * L, L)]
                        valid = dl < nq
                        dl = jnp.where(valid, dl, 0)
                        gidx2_v[buf, pl.ds(j * L, L)] = s
                        dloc2_v[buf, pl.ds(j * L, L)] = dl
                        hs16 = plsc.load_gather(hs_v, [s])
                        hd16 = plsc.load_gather(hdq_v, [dl])
                        ee = hs16 + hd16
                        w = jnp.exp(jnp.where(ee >= 0, ee, ALPHA * ee))
                        w = jnp.where(valid, w, 0.0)
                        rr = lax.shift_right_logical(dl, 7)
                        cc = lax.bitwise_and(dl, 127)
                        plsc.addupdate_scatter(den_v, [rr, cc], w)
                        w_v[buf, pl.ds(j * L, L)] = w

                def issue(buf):
                    pltpu.async_copy(h_hbm.at[gidx2_v.at[buf]],
                                     rows_v.at[buf], sem_g[buf])

                def wait_gather(buf):
                    pltpu.make_async_copy(h_hbm.at[gidx2_v.at[buf]],
                                          rows_v.at[buf], sem_g[buf]).wait()

                def scale_scatter(buf):
                    for g in range(eb // L):
                        wv = w_v[buf, pl.ds(g * L, L)]
                        for lane in range(L):
                            wr = wv[lane]
                            r = g * L + lane
                            for c in range(8):
                                rows_v[buf, r, pl.ds(c * L, L)] = (
                                    rows_v[buf, r, pl.ds(c * L, L)] * wr)
                    pltpu.async_copy(rows_v.at[buf],
                                     acc_spm.at[dloc2_v.at[buf]],
                                     sem_sc[buf], add=True)

                def wait_scat(buf):
                    pltpu.make_async_copy(rows_v.at[buf],
                                          acc_spm.at[dloc2_v.at[buf]],
                                          sem_sc[buf]).wait()

                @pl.when(nblk > 0)
                def _pro0():
                    prep(0, 0)
                    issue(0)

                @pl.when(nblk > 1)
                def _pro1():
                    prep(1, 1)
                    issue(1)

                def tri_body(i, _):
                    for k in range(3):
                        b = 3 * i + k
                        bufn = (k + 2) % 3

                        @pl.when(b + 2 < nblk)
                        def _pf():
                            prep(b + 2, bufn)

                            @pl.when(b >= 1)
                            def _ws():
                                wait_scat(bufn)

                            issue(bufn)

                        @pl.when(b < nblk)
                        def _pr():
                            wait_gather(k)
                            scale_scatter(k)

                    return 0

                lax.fori_loop(0, lax.div(nblk + 2, 3), tri_body, 0)

                for k in range(3):

                    @pl.when(nblk > k)
                    def _drain():
                        wait_scat(k)

                return 0


            lax.fori_loop(0, epc // chunk, chunk_body, 0)

            pltpu.async_copy(den_v, denacc_spm.at[iden_v], sem_s,
                             add=True).wait()
            plsc.subcore_barrier()

            rstart = jnp.minimum(sid * q, nq - q)
            pltpu.sync_copy(acc_spm.at[pl.ds(rstart, q)],
                            num_hbm.at[pl.ds(base + rstart, q)])
            dstart = jnp.minimum(sid * dq, ndr - dq)
            pltpu.sync_copy(denacc_spm.at[pl.ds(dstart, dq)],
                            den_hbm.at[ph, cid, pl.ds(dstart, dq)])
            plsc.subcore_barrier()
            return 0

        lax.fori_loop(0, 2, phase_body, 0)

    num_p, den_p = k(h_p, hs, hd, src_p, dst_p)
    num = num_p[:n]
    den = den_p.reshape(2 * NC, ndr * 128)[:, :nq].reshape(-1)[:n]
    return num, den.reshape(n, 1)


def _pair_gather(x1, x2, idx1, idx2):
    b = idx1.shape[0]
    r = b // NW

    @functools.partial(
        pl.kernel,
        out_type=[
            jax.ShapeDtypeStruct((b, 128), jnp.float32),
            jax.ShapeDtypeStruct((b, 128), jnp.float32),
        ],
        mesh=plsc.VectorSubcoreMesh(**_MESH),
        compiler_params=pltpu.CompilerParams(needs_layout_passes=False),
        scratch_types=[
            pltpu.VMEM((r,), jnp.int32),
            pltpu.VMEM((r, 128), jnp.float32),
            pltpu.SemaphoreType.DMA,
        ],
    )
    def k(x1_hbm, x2_hbm, i1_hbm, i2_hbm, o1_hbm, o2_hbm, idx_v, rows_v, sem):
        wid = lax.axis_index("s") * NC + lax.axis_index("c")
        base = wid * r
        pltpu.sync_copy(i1_hbm.at[pl.ds(base, r)], idx_v)
        pltpu.async_copy(x1_hbm.at[idx_v], rows_v, sem).wait()
        pltpu.sync_copy(rows_v, o1_hbm.at[pl.ds(base, r)])
        pltpu.sync_copy(i2_hbm.at[pl.ds(base, r)], idx_v)
        pltpu.async_copy(x2_hbm.at[idx_v], rows_v, sem).wait()
        pltpu.sync_copy(rows_v, o2_hbm.at[pl.ds(base, r)])

    return k(x1, x2, idx1, idx2)


def _spgat(x, den0, src, dst, W1p, a1s, a1d, W2, a2s, a2d, n, weight_W,
           weight_proj):
    h1, hsd1 = _mm_hsd(x, den0, W1p, a1s, a1d, layer2=False)
    num1, den1 = _edge_aggregate(h1, hsd1, src, dst, n)
    h2, hsd2 = _mm_hsd(num1, den1, W2, a2s, a2d, layer2=True)
    num2, den2 = _edge_aggregate(h2, hsd2, src, dst, n)
    return _final_graph(num2, den2, weight_W, weight_proj)


def kernel(feat_idx, tw_src, tw_dst, ut_src, ut_dst, tw_graph_idx, ut_graph_idx,
           word_embedding, user_embedding, tw_W1, tw_a1s, tw_a1d, tw_W2, tw_a2s,
           tw_a2d, ut_W1, ut_a1s, ut_a1d, ut_W2, ut_a2s, ut_a2d, weight_W,
           weight_proj, out_W, out_b):
    n_tw, words = feat_idx.shape
    n_ut = user_embedding.shape[0]

    feat_flat = feat_idx.reshape(-1).astype(jnp.int32)
    twt_X = _embed_mean(word_embedding, feat_flat, n_tw, words)

    one_tw = jnp.ones((n_tw, 1), jnp.float32)
    one_ut = jnp.ones((n_ut, 1), jnp.float32)
    tw_X, tw_psum = _spgat(twt_X, one_tw, tw_src.astype(jnp.int32),
                           tw_dst.astype(jnp.int32), tw_W1, tw_a1s, tw_a1d,
                           tw_W2, tw_a2s, tw_a2d, n_tw, weight_W, weight_proj)
    tu_X, tu_psum = _spgat(user_embedding, one_ut, ut_src.astype(jnp.int32),
                           ut_dst.astype(jnp.int32), ut_W1, ut_a1s, ut_a1d,
                           ut_W2, ut_a2s, ut_a2d, n_ut, weight_W, weight_proj)

    att_tw = tw_psum[0, 0] / n_tw
    att_tu = tu_psum[0, 0] / n_ut
    m = jnp.maximum(att_tw, att_tu)
    e0 = jnp.exp(att_tw - m)
    e1 = jnp.exp(att_tu - m)
    att = jnp.stack([e0, e1]) / (e0 + e1)

    g1, g2 = _pair_gather(tw_X, tu_X, tw_graph_idx.astype(jnp.int32),
                          ut_graph_idx.astype(jnp.int32))

    nclass = out_W.shape[0]
    w_pad = jnp.pad(out_W.T, ((0, 0), (0, 128 - nclass)))
    b_pad = jnp.pad(out_b[None, :], ((0, 0), (0, 128 - nclass)),
                    constant_values=NEG)
    lp = _logits(g1, g2, att, w_pad, b_pad)
    return lp[:, :nclass]


# speedup vs baseline: 1.2294x; 1.0023x over previous
import functools

import jax
import jax.numpy as jnp
from jax import lax
from jax.experimental import pallas as pl
from jax.experimental.pallas import tpu as pltpu
from jax.experimental.pallas import tpu_sc as plsc

ALPHA = 0.2
EPS = 1e-16
NEG = -1e9

NC = 2
NS = 16
NW = NC * NS
L = 16

_MESH = dict(core_axis_name="c", subcore_axis_name="s", num_cores=NC,
             num_subcores=NS)


def _cdiv(a, b):
    return (a + b - 1) // b


def _mm_hsd_body(num_ref, den_ref, w_ref, asd_ref, h_ref, hsd_ref, *, layer2):
    x = num_ref[...]
    if layer2:
        x = x / (den_ref[...] + EPS)
        x = jnp.where(x > 0, x, jnp.exp(x) - 1.0)
    h = jnp.dot(x, w_ref[...], preferred_element_type=jnp.float32)
    h_ref[...] = h
    hsd_ref[...] = jnp.dot(h, asd_ref[...], preferred_element_type=jnp.float32)


def _mm_hsd(x, den, W, a_s, a_d, *, layer2, blk=400):
    n, k = x.shape
    asd = jnp.stack([a_s, a_d], axis=1)
    kfn = functools.partial(_mm_hsd_body, layer2=layer2)
    return pl.pallas_call(
        kfn,
        grid=(n // blk,),
        in_specs=[
            pl.BlockSpec((blk, k), lambda i: (i, 0)),
            pl.BlockSpec((blk, 1), lambda i: (i, 0)),
            pl.BlockSpec((k, 128), lambda i: (0, 0)),
            pl.BlockSpec((128, 2), lambda i: (0, 0)),
        ],
        out_specs=[
            pl.BlockSpec((blk, 128), lambda i: (i, 0)),
            pl.BlockSpec((blk, 2), lambda i: (i, 0)),
        ],
        out_shape=[
            jax.ShapeDtypeStruct((n, 128), jnp.float32),
            jax.ShapeDtypeStruct((n, 2), jnp.float32),
        ],
    )(x, den, W, asd)


def _final_graph_body(num_ref, den_ref, w_ref, proj_ref, x_out_ref, psum_ref):
    i = pl.program_id(0)
    x = num_ref[...] / (den_ref[...] + EPS)
    x = jnp.where(x > 0, x, jnp.exp(x) - 1.0)
    x_out_ref[...] = x
    u = jnp.tanh(jnp.dot(x, w_ref[...], preferred_element_type=jnp.float32))
    p = jnp.dot(u, proj_ref[...], preferred_element_type=jnp.float32)

    @pl.when(i == 0)
    def _init():
        psum_ref[...] = jnp.zeros_like(psum_ref)

    psum_ref[...] = psum_ref[...] + jnp.sum(p)


def _final_graph(num, den, weight_W, weight_proj, *, blk=400):
    n = num.shape[0]
    return pl.pallas_call(
        _final_graph_body,
        grid=(n // blk,),
        in_specs=[
            pl.BlockSpec((blk, 128), lambda i: (i, 0)),
            pl.BlockSpec((blk, 1), lambda i: (i, 0)),
            pl.BlockSpec((128, 128), lambda i: (0, 0)),
            pl.BlockSpec((128, 1), lambda i: (0, 0)),
        ],
        out_specs=[
            pl.BlockSpec((blk, 128), lambda i: (i, 0)),
            pl.BlockSpec((1, 1), lambda i: (0, 0)),
        ],
        out_shape=[
            jax.ShapeDtypeStruct((n, 128), jnp.float32),
            jax.ShapeDtypeStruct((1, 1), jnp.float32),
        ],
    )(num, den, weight_W, weight_proj)


def _logits_body(g1_ref, g2_ref, a_ref, w_ref, b_ref, out_ref):
    a0 = a_ref[0]
    a1 = a_ref[1]
    of = a0 * g1_ref[...] + a1 * g2_ref[...]
    z = jnp.dot(of, w_ref[...], preferred_element_type=jnp.float32) + b_ref[...]
    m = jnp.max(z, axis=1, keepdims=True)
    z = z - m
    out_ref[...] = z - jnp.log(jnp.sum(jnp.exp(z), axis=1, keepdims=True))


def _logits(g1, g2, att, w_pad, b_pad, *, blk=512):
    b = g1.shape[0]
    return pl.pallas_call(
        _logits_body,
        grid=(b // blk,),
        in_specs=[
            pl.BlockSpec((blk, 128), lambda i: (i, 0)),
            pl.BlockSpec((blk, 128), lambda i: (i, 0)),
            pl.BlockSpec(memory_space=pltpu.SMEM),
            pl.BlockSpec((128, 128), lambda i: (0, 0)),
            pl.BlockSpec((1, 128), lambda i: (0, 0)),
        ],
        out_specs=pl.BlockSpec((blk, 128), lambda i: (i, 0)),
        out_shape=jax.ShapeDtypeStruct((b, 128), jnp.float32),
    )(g1, g2, att, w_pad, b_pad)


def _embed_mean(word_emb, feat_flat, n, words):
    kc = word_emb.shape[1]
    tb = n // L
    nb = _cdiv(tb, NW)
    cols = list(range(0, kc - L + 1, L))
    if cols[-1] != kc - L:
        cols.append(kc - L)

    @functools.partial(
        pl.kernel,
        out_type=jax.ShapeDtypeStruct((n, kc), jnp.float32),
        mesh=plsc.VectorSubcoreMesh(**_MESH),
        compiler_params=pltpu.CompilerParams(needs_layout_passes=False,
                                             use_tc_tiling_on_sc=False),
        scratch_types=[
            pltpu.VMEM((L * words,), jnp.int32),
            pltpu.VMEM((L * words,), jnp.int32),
            pltpu.VMEM((2, L * words, kc), jnp.float32),
            pltpu.VMEM((L, kc), jnp.float32),
            pltpu.SemaphoreType.DMA,
            pltpu.SemaphoreType.DMA,
        ],
    )
    def k(we_hbm, fi_hbm, out_hbm, fidx_a, fidx_b, rows_v, obuf_v, sem_a,
          sem_b):
        wid = lax.axis_index("s") * NC + lax.axis_index("c")

        def jof(b):
            return jnp.minimum(wid * nb + b, tb - 1)

        def load_issue(b, buf, sem):
            j = jof(b)
            fidx = fidx_a if buf == 0 else fidx_b
            pltpu.sync_copy(fi_hbm.at[pl.ds(j * (L * words), L * words)],
                            fidx)
            return pltpu.async_copy(we_hbm.at[fidx], rows_v.at[buf], sem)

        def compute_out(b, buf):
            def node(kk, _):
                base = kk * words
                for c in cols:
                    a = rows_v[buf, base, pl.ds(c, L)]
                    for r in range(1, words):
                        a = a + rows_v[buf, base + r, pl.ds(c, L)]
                    obuf_v[kk, pl.ds(c, L)] = a * (1.0 / words)
                return 0

            lax.fori_loop(0, L, node, 0)
            pltpu.sync_copy(obuf_v, out_hbm.at[pl.ds(jof(b) * L, L)])

        load_issue(0, 0, sem_a)

        def pair(i, _):
            ba = 2 * i
            bb = ba + 1

            @pl.when(bb < nb)
            def _ib():
                load_issue(bb, 1, sem_b)

            pltpu.make_async_copy(we_hbm.at[fidx_a], rows_v.at[0],
                                  sem_a).wait()
            compute_out(ba, 0)

            @pl.when(ba + 2 < nb)
            def _ia():
                load_issue(ba + 2, 0, sem_a)

            @pl.when(bb < nb)
            def _cb():
                pltpu.make_async_copy(we_hbm.at[fidx_b], rows_v.at[1],
                                      sem_b).wait()
                compute_out(bb, 1)

            return 0

        lax.fori_loop(0, _cdiv(nb, 2), pair, 0)

    return k(word_emb, feat_flat)


def _edge_aggregate(h, hsd, src, dst, n):
    e = src.shape[0]
    chunk = 2048
    n_pad = _cdiv(n, 32) * 32
    e_pad = _cdiv(e, NS * chunk) * (NS * chunk)
    eb = 32
    epc = e_pad // NS
    nq = n_pad // 4
    nqp = nq
    ndr = _cdiv(_cdiv(nq + 1, 128), L) * L
    q = _cdiv(_cdiv(nq, NS), 8) * 8
    zc = _cdiv(nqp, NS * eb)
    z = zc * eb
    dq = 8

    src_p = jnp.pad(src, (0, e_pad - e))
    dst_p = jnp.pad(dst, (0, e_pad - e), constant_values=n_pad)
    if n_pad == n:
        h_p = h
        hs = hsd[:, 0:1].reshape(n)
        hd = hsd[:, 1:2].reshape(n)
    else:
        h_p = jnp.pad(h, ((0, n_pad - n), (0, 0)))
        hs = jnp.pad(hsd[:, 0:1].reshape(n), (0, n_pad - n))
        hd = jnp.pad(hsd[:, 1:2].reshape(n), (0, n_pad - n))

    @functools.partial(
        pl.kernel,
        out_type=[
            jax.ShapeDtypeStruct((n_pad, 128), jnp.float32),
            jax.ShapeDtypeStruct((2, NC, ndr, 128), jnp.float32),
        ],
        mesh=plsc.VectorSubcoreMesh(**_MESH),
        compiler_params=pltpu.CompilerParams(needs_layout_passes=False),
        scratch_types=[
            pltpu.VMEM((n_pad,), jnp.float32),
            pltpu.VMEM((nq,), jnp.float32),
            pltpu.VMEM((ndr, 128), jnp.float32),
            pltpu.VMEM((ndr,), jnp.int32),
            pltpu.VMEM((chunk,), jnp.int32),
            pltpu.VMEM((chunk,), jnp.int32),
            pltpu.VMEM((chunk + eb,), jnp.int32),
            pltpu.VMEM((chunk + eb,), jnp.int32),
            pltpu.VMEM((3, eb), jnp.int32),
            pltpu.VMEM((3, eb), jnp.int32),
            pltpu.VMEM((3, eb), jnp.float32),
            pltpu.VMEM((3, eb, 128), jnp.float32),
            pltpu.VMEM_SHARED((nqp, 128), jnp.float32),
            pltpu.VMEM_SHARED((ndr, 128), jnp.float32),
            [pltpu.SemaphoreType.DMA] * 3,
            [pltpu.SemaphoreType.DMA] * 3,
            pltpu.SemaphoreType.DMA,
        ],
    )
    def k(h_hbm, hs_hbm, hd_hbm, src_hbm, dst_hbm, num_hbm, den_hbm,
          hs_v, hdq_v, den_v, iden_v, src_v, dst_v, sc_v, dc_v,
          gidx2_v, dloc2_v, w_v, rows_v, acc_spm, denacc_spm, sem_g,
          sem_sc, sem_s):
        cid = lax.axis_index("c")
        sid = lax.axis_index("s")
        zero = jnp.zeros((L,), jnp.float32)
        ramp = lax.iota(jnp.int32, L)

        pltpu.sync_copy(hs_hbm, hs_v)

        for i in range(ndr // L):
            iden_v[pl.ds(i * L, L)] = ramp + (i * L)

        def phase_body(ph, _):
            base = pl.multiple_of(ph * (2 * nq) + cid * nq, 8)
            pltpu.sync_copy(hd_hbm.at[pl.ds(base, nq)], hdq_v)

            def zden(i, _):
                for c in range(8):
                    den_v[i, pl.ds(c * L, L)] = zero
                return 0

            lax.fori_loop(0, ndr, zden, 0)

            def zrow(r, _):
                for c in range(8):
                    rows_v[0, r, pl.ds(c * L, L)] = zero
                return 0

            lax.fori_loop(0, eb, zrow, 0)

            zstart = jnp.minimum(sid * z, nqp - z)

            def zacc(i, _):
                pltpu.sync_copy(rows_v.at[0], acc_spm.at[pl.ds(zstart + i * eb, eb)])
                return 0

            lax.fori_loop(0, zc, zacc, 0)

            @pl.when(sid == 0)
            def _zden_shared():
                pltpu.sync_copy(den_v, denacc_spm)

            plsc.subcore_barrier()

            def chunk_body(ci, _):
                eoff = sid * epc + ci * chunk
                pltpu.async_copy(src_hbm.at[pl.ds(eoff, chunk)], src_v,
                                 sem_s)
                pltpu.async_copy(dst_hbm.at[pl.ds(eoff, chunk)], dst_v,
                                 sem_s)
                pltpu.make_async_copy(src_hbm.at[pl.ds(eoff, chunk)], src_v,
                                      sem_s).wait()
                pltpu.make_async_copy(dst_hbm.at[pl.ds(eoff, chunk)], dst_v,
                                      sem_s).wait()

                def pre(i, _):
                    sc_v[pl.ds(i * L, L)] = ramp * 0
                    dc_v[pl.ds(i * L, L)] = ramp * 0 + nq
                    return 0

                lax.fori_loop(0, (chunk + eb) // L, pre, 0)

                def cmp_body(j, cnt):
                    s = src_v[pl.ds(j * L, L)]
                    d = dst_v[pl.ds(j * L, L)]
                    inq = (d >= base) & (d < base + nq)
                    dloc = d - base
                    csum = plsc.cumsum(inq.astype(jnp.int32))
                    idx = cnt + csum - 1
                    plsc.store_scatter(sc_v, [idx], s, mask=inq)
                    plsc.store_scatter(dc_v, [idx], dloc, mask=inq)
                    return cnt + csum[L - 1]

                cnt = lax.fori_loop(0, chunk // L, cmp_body, 0)
                nblk = lax.div(cnt + (eb - 1), eb)

                def prep(b, buf):
                    boff = b * eb
                    for j in range(eb // L):
                        s = sc_v[pl.ds(boff + j * L, L)]
                        dl = dc_v[pl.ds(boff + j * L, L)]
                        valid = dl < nq
                        dl = jnp.where(valid, dl, 0)
                        gidx2_v[buf, pl.ds(j * L, L)] = s
                        dloc2_v[buf, pl.ds(j * L, L)] = dl
                        hs16 = plsc.load_gather(hs_v, [s])
                        hd16 = plsc.load_gather(hdq_v, [dl])
                        ee = hs16 + hd16
                        w = jnp.exp(jnp.where(ee >= 0, ee, ALPHA * ee))
                        w = jnp.where(valid, w, 0.0)
                        rr = lax.shift_right_logical(dl, 7)
                        cc = lax.bitwise_and(dl, 127)
                        plsc.addupdate_scatter(den_v, [rr, cc], w)
                        w_v[buf, pl.ds(j * L, L)] = w

                def issue(buf):
                    pltpu.async_copy(h_hbm.at[gidx2_v.at[buf]],
                                     rows_v.at[buf], sem_g[buf])

                def wait_gather(buf):
                    pltpu.make_async_copy(h_hbm.at[gidx2_v.at[buf]],
                                          rows_v.at[buf], sem_g[buf]).wait()

                def scale_scatter(buf):
                    for g in range(eb // L):
                        wv = w_v[buf, pl.ds(g * L, L)]
                        for lane in range(L):
                            wr = wv[lane]
                            r = g * L + lane
                            for c in range(8):
                                rows_v[buf, r, pl.ds(c * L, L)] = (
                                    rows_v[buf, r, pl.ds(c * L, L)] * wr)
                    pltpu.async_copy(rows_v.at[buf],
                                     acc_spm.at[dloc2_v.at[buf]],
                                     sem_sc[buf], add=True)

                def wait_scat(buf):
                    pltpu.make_async_copy(rows_v.at[buf],
                                          acc_spm.at[dloc2_v.at[buf]],
                                          sem_sc[buf]).wait()

                @pl.when(nblk > 0)
                def _pro0():
                    prep(0, 0)
                    issue(0)

                @pl.when(nblk > 1)
                def _pro1():
                    prep(1, 1)
                    issue(1)

                def tri_body(i, _):
                    for k in range(3):
                        b = 3 * i + k
                        bufn = (k + 2) % 3

                        @pl.when(b + 2 < nblk)
                        def _pf():
                            prep(b + 2, bufn)

                            @pl.when(b >= 1)
                            def _ws():
                                wait_scat(bufn)

                            issue(bufn)

                        @pl.when(b < nblk)
                        def _pr():
                            wait_gather(k)
                            scale_scatter(k)

                    return 0

                lax.fori_loop(0, lax.div(nblk + 2, 3), tri_body, 0)

                for k in range(3):

                    @pl.when(nblk > k)
                    def _drain():
                        wait_scat(k)

                return 0


            lax.fori_loop(0, epc // chunk, chunk_body, 0)

            pltpu.async_copy(den_v, denacc_spm.at[iden_v], sem_s,
                             add=True).wait()
            plsc.subcore_barrier()

            rstart = jnp.minimum(sid * q, nq - q)
            pltpu.sync_copy(acc_spm.at[pl.ds(rstart, q)],
                            num_hbm.at[pl.ds(base + rstart, q)])
            dstart = jnp.minimum(sid * dq, ndr - dq)
            pltpu.sync_copy(denacc_spm.at[pl.ds(dstart, dq)],
                            den_hbm.at[ph, cid, pl.ds(dstart, dq)])
            plsc.subcore_barrier()
            return 0

        lax.fori_loop(0, 2, phase_body, 0)

    num_p, den_p = k(h_p, hs, hd, src_p, dst_p)
    num = num_p[:n]
    den = den_p.reshape(2 * NC, ndr * 128)[:, :nq].reshape(-1)[:n]
    return num, den.reshape(n, 1)


def _pair_gather(x1, x2, idx1, idx2):
    b = idx1.shape[0]
    r = b // NW

    @functools.partial(
        pl.kernel,
        out_type=[
            jax.ShapeDtypeStruct((b, 128), jnp.float32),
            jax.ShapeDtypeStruct((b, 128), jnp.float32),
        ],
        mesh=plsc.VectorSubcoreMesh(**_MESH),
        compiler_params=pltpu.CompilerParams(needs_layout_passes=False),
        scratch_types=[
            pltpu.VMEM((r,), jnp.int32),
            pltpu.VMEM((r, 128), jnp.float32),
            pltpu.SemaphoreType.DMA,
        ],
    )
    def k(x1_hbm, x2_hbm, i1_hbm, i2_hbm, o1_hbm, o2_hbm, idx_v, rows_v, sem):
        wid = lax.axis_index("s") * NC + lax.axis_index("c")
        base = wid * r
        pltpu.sync_copy(i1_hbm.at[pl.ds(base, r)], idx_v)
        pltpu.async_copy(x1_hbm.at[idx_v], rows_v, sem).wait()
        pltpu.sync_copy(rows_v, o1_hbm.at[pl.ds(base, r)])
        pltpu.sync_copy(i2_hbm.at[pl.ds(base, r)], idx_v)
        pltpu.async_copy(x2_hbm.at[idx_v], rows_v, sem).wait()
        pltpu.sync_copy(rows_v, o2_hbm.at[pl.ds(base, r)])

    return k(x1, x2, idx1, idx2)


def _spgat(x, den0, src, dst, W1p, a1s, a1d, W2, a2s, a2d, n, weight_W,
           weight_proj):
    h1, hsd1 = _mm_hsd(x, den0, W1p, a1s, a1d, layer2=False)
    num1, den1 = _edge_aggregate(h1, hsd1, src, dst, n)
    h2, hsd2 = _mm_hsd(num1, den1, W2, a2s, a2d, layer2=True)
    num2, den2 = _edge_aggregate(h2, hsd2, src, dst, n)
    return _final_graph(num2, den2, weight_W, weight_proj)


def kernel(feat_idx, tw_src, tw_dst, ut_src, ut_dst, tw_graph_idx, ut_graph_idx,
           word_embedding, user_embedding, tw_W1, tw_a1s, tw_a1d, tw_W2, tw_a2s,
           tw_a2d, ut_W1, ut_a1s, ut_a1d, ut_W2, ut_a2s, ut_a2d, weight_W,
           weight_proj, out_W, out_b):
    n_tw, words = feat_idx.shape
    n_ut = user_embedding.shape[0]

    feat_flat = feat_idx.reshape(-1).astype(jnp.int32)
    twt_X = _embed_mean(word_embedding, feat_flat, n_tw, words)

    one_tw = jnp.ones((n_tw, 1), jnp.float32)
    one_ut = jnp.ones((n_ut, 1), jnp.float32)
    tw_X, tw_psum = _spgat(twt_X, one_tw, tw_src.astype(jnp.int32),
                           tw_dst.astype(jnp.int32), tw_W1, tw_a1s, tw_a1d,
                           tw_W2, tw_a2s, tw_a2d, n_tw, weight_W, weight_proj)
    tu_X, tu_psum = _spgat(user_embedding, one_ut, ut_src.astype(jnp.int32),
                           ut_dst.astype(jnp.int32), ut_W1, ut_a1s, ut_a1d,
                           ut_W2, ut_a2s, ut_a2d, n_ut, weight_W, weight_proj)

    att_tw = tw_psum[0, 0] / n_tw
    att_tu = tu_psum[0, 0] / n_ut
    m = jnp.maximum(att_tw, att_tu)
    e0 = jnp.exp(att_tw - m)
    e1 = jnp.exp(att_tu - m)
    att = jnp.stack([e0, e1]) / (e0 + e1)

    g1, g2 = _pair_gather(tw_X, tu_X, tw_graph_idx.astype(jnp.int32),
                          ut_graph_idx.astype(jnp.int32))

    nclass = out_W.shape[0]
    w_pad = jnp.pad(out_W.T, ((0, 0), (0, 128 - nclass)))
    b_pad = jnp.pad(out_b[None, :], ((0, 0), (0, 128 - nclass)),
                    constant_values=NEG)
    lp = _logits(g1, g2, att, w_pad, b_pad)
    return lp[:, :nclass]
